# baseline (speedup 1.0000x reference)
import functools
import math

import jax
import jax.numpy as jnp
from jax import lax
from jax.experimental import pallas as pl
from jax.experimental.pallas import tpu as pltpu
from jax.experimental.pallas import tpu_sc as plsc

NCELL = 50000
NNET = 50000
NPIN = 800000
NPAIR = 400000

NC, NS, L = 2, 16, 16
NW = NC * NS

NPAD = 50176
PASSES = 4
SLICE = 392
EPIN = 819200
CH = 2048
NCHUNK = EPIN // CH
MB = 128
FLUSH_AT = MB - 16
PPAIR = 409600

_mesh = plsc.VectorSubcoreMesh(core_axis_name="c", subcore_axis_name="s")


def _wid():
    return lax.axis_index("s") * NC + lax.axis_index("c")


def _f32(shape):
    return jax.ShapeDtypeStruct(shape, jnp.float32)


def _seg_body(dsrc_hbm, table_hbm, sum_hbm, max_hbm, cnt_hbm,
              sumtbl, maxtbl, cnttbl, dsrcbuf, mrel, msrc, rows,
              sem0, sem1, gsem):
    wid = _wid()
    zero16 = jnp.zeros((L,), jnp.float32)
    ninf16 = jnp.full((L,), -jnp.inf, jnp.float32)
    iota = lax.broadcasted_iota(jnp.int32, (L,), 0)

    def init_msrc(i, _):
        msrc[pl.ds(i * L, L)] = (wid * 251 + i * L + iota) % NCELL
        return 0
    lax.fori_loop(0, MB // L, init_msrc, 0)

    lane0 = iota == 0
    one16 = jnp.ones((L,), jnp.float32)

    def flush(count):
        pltpu.async_copy(table_hbm.at[msrc.at[pl.ds(0, MB)]], rows, gsem).wait()

        def acc(r, _):
            dl = mrel[pl.ds(r, L)][0]
            off = dl * 128
            for j in range(8):
                g = rows[r, pl.ds(j * L, L)]
                sl = pl.ds(off + j * L, L)
                sumtbl[sl] = sumtbl[sl] + g
                maxtbl[sl] = jnp.maximum(maxtbl[sl], g)
            plsc.addupdate_scatter(cnttbl, [jnp.full((L,), dl, jnp.int32)],
                                   one16, mask=lane0)
            return 0
        lax.fori_loop(0, count, acc, 0)

    def do_pass(p, _):
        base = (wid * PASSES + p) * SLICE

        def initrow(i, _):
            sumtbl[pl.ds(i * L, L)] = zero16
            maxtbl[pl.ds(i * L, L)] = ninf16
            return 0
        lax.fori_loop(0, SLICE * 128 // L, initrow, 0)

        def initcnt(i, _):
            cnttbl[pl.ds(i * L, L)] = zero16
            return 0
        lax.fori_loop(0, SLICE // 8 // 2 + 1, initcnt, 0)

        def start_load(c, slot):
            pltpu.async_copy(dsrc_hbm.at[c], dsrcbuf.at[slot],
                             sem0 if slot == 0 else sem1)

        def wait_load(slot):
            pltpu.make_async_copy(dsrc_hbm.at[0], dsrcbuf.at[slot],
                                  sem0 if slot == 0 else sem1).wait()

        def scan_chunk(slot, cursor):
            def scan_vec(v, cur):
                d = dsrcbuf[slot, 0, pl.ds(v * L, L)]
                rel = d - base
                m = (rel >= 0) & (rel < SLICE)
                s = dsrcbuf[slot, 1, pl.ds(v * L, L)]
                key = jnp.where(m, rel, jnp.int32(0x7FFFFFFF))
                sk, sv = plsc.sort_key_val(key, s)
                mrel[pl.ds(cur, L)] = sk
                msrc[pl.ds(cur, L)] = sv
                n = plsc.all_reduce_population_count(m)[0]
                cur2 = cur + n
                pred = cur2 >= FLUSH_AT

                @pl.when(pred)
                def _():
                    flush(cur2)
                return jnp.where(pred, 0, cur2)
            return lax.fori_loop(0, CH // L, scan_vec, cursor)

        start_load(0, 0)

        def chunk_pair(i, cursor):
            start_load(2 * i + 1, 1)
            wait_load(0)
            cursor = scan_chunk(0, cursor)

            @pl.when(2 * i + 2 < NCHUNK)
            def _():
                start_load(2 * i + 2, 0)
            wait_load(1)
            cursor = scan_chunk(1, cursor)
            return cursor
        cursor = lax.fori_loop(0, NCHUNK // 2, chunk_pair, 0)

        @pl.when(cursor > 0)
        def _():
            flush(cursor)

        pltpu.sync_copy(sumtbl, sum_hbm.at[pl.ds(base * 128, SLICE * 128)])
        pltpu.sync_copy(maxtbl, max_hbm.at[pl.ds(base * 128, SLICE * 128)])
        pltpu.sync_copy(cnttbl.at[pl.ds(0, SLICE)],
                        cnt_hbm.at[pl.ds(base, SLICE)])
        return 0
    lax.fori_loop(0, PASSES, do_pass, 0)


_seg_call = pl.kernel(
    _seg_body,
    out_type=[_f32((NPAD * 128,)), _f32((NPAD * 128,)), _f32((NPAD,))],
    mesh=_mesh,
    compiler_params=pltpu.CompilerParams(needs_layout_passes=False),
    scratch_types=[
        pltpu.VMEM((SLICE * 128,), jnp.float32),
        pltpu.VMEM((SLICE * 128,), jnp.float32),
        pltpu.VMEM((SLICE + 8,), jnp.float32),
        pltpu.VMEM((2, 2, CH), jnp.int32),
        pltpu.VMEM((MB + L,), jnp.int32),
        pltpu.VMEM((MB + L,), jnp.int32),
        pltpu.VMEM((MB, 128), jnp.float32),
        pltpu.SemaphoreType.DMA,
        pltpu.SemaphoreType.DMA,
        pltpu.SemaphoreType.DMA,
    ],
)


RPT = NPAD // NS
ROWCH = 8
TROWS = EPIN // 128 // NW


def _wseg_body(pn_hbm, pc_hbm, ew_hbm, u_hbm, wacc_hbm,
               nbuf, cbuf, ebuf, urowsA, urowsB, zbuf, shared,
               lsem, gsemA, gsemB):
    wid = _wid()
    sid = lax.axis_index("s")
    cid = lax.axis_index("c")
    zero16 = jnp.zeros((L,), jnp.float32)

    def initz(i, _):
        zbuf[i, :] = zero16
        return 0
    lax.fori_loop(0, RPT // NS, initz, 0)

    def initsh(k, _):
        pltpu.sync_copy(zbuf, shared.at[pl.ds(sid * RPT + k * (RPT // NS),
                                              RPT // NS), :])
        return 0
    lax.fori_loop(0, NS, initsh, 0)
    plsc.subcore_barrier()

    def chunk(ci, _):
        rowbase = wid * TROWS + ci * ROWCH
        pltpu.async_copy(pn_hbm.at[pl.ds(rowbase, ROWCH), :], nbuf, lsem)
        pltpu.async_copy(pc_hbm.at[pl.ds(rowbase, ROWCH), :], cbuf, lsem)
        pltpu.async_copy(ew_hbm.at[pl.ds(rowbase, ROWCH), :], ebuf, lsem)
        for _ in range(3):
            pltpu.make_async_copy(pn_hbm.at[pl.ds(0, ROWCH), :], nbuf,
                                  lsem).wait()

        pltpu.async_copy(u_hbm.at[nbuf.at[0]], urowsA, gsemA)
        for k in range(ROWCH):
            cur, csem = (urowsA, gsemA) if k % 2 == 0 else (urowsB, gsemB)
            nxt, nsem = (urowsB, gsemB) if k % 2 == 0 else (urowsA, gsemA)
            if k < ROWCH - 1:
                pltpu.async_copy(u_hbm.at[nbuf.at[k + 1]], nxt, nsem)
            pltpu.make_async_copy(u_hbm.at[nbuf.at[k]], cur, csem).wait()

            def scale(g, _):
                ev = ebuf[k, pl.ds(g * L, L)]
                for j in range(L):
                    r = g * L + j
                    cur[r, :] = cur[r, :] * ev[j]
                return 0
            lax.fori_loop(0, 128 // L, scale, 0)
            pltpu.sync_copy(cur, shared.at[cbuf.at[k]], add=True)
        return 0
    lax.fori_loop(0, TROWS // ROWCH, chunk, 0)

    plsc.subcore_barrier()
    pltpu.sync_copy(shared.at[pl.ds(sid * RPT, RPT), :],
                    wacc_hbm.at[cid, pl.ds(sid * RPT, RPT), :])


_wseg_call = pl.kernel(
    _wseg_body,
    out_type=[_f32((NC, NPAD, L))],
    mesh=_mesh,
    compiler_params=pltpu.CompilerParams(needs_layout_passes=False, use_tc_tiling_on_sc=False),
    scratch_types=[
        pltpu.VMEM((ROWCH, 128), jnp.int32),
        pltpu.VMEM((ROWCH, 128), jnp.int32),
        pltpu.VMEM((ROWCH, 128), jnp.float32),
        pltpu.VMEM((128, L), jnp.float32),
        pltpu.VMEM((128, L), jnp.float32),
        pltpu.VMEM((RPT // NS, L), jnp.float32),
        pltpu.VMEM_SHARED((NPAD, L), jnp.float32),
        pltpu.SemaphoreType.DMA,
        pltpu.SemaphoreType.DMA,
        pltpu.SemaphoreType.DMA,
    ],
)


PROWS = PPAIR // 128 // NW
TWO_PI = 2.0 * math.pi


def _pair_body(fa_hbm, so_hbm, gf_hbm, fn_hbm, gn_hbm,
               tcell_hbm, tnet_hbm, darr_hbm, garr_hbm,
               o1_hbm, o2_hbm,
               fab, sob, gfb, fnb, gnb,
               rfA, rsA, rnA, dvA, gvA, rfB, rsB, rnB, dvB, gvB,
               ob1, ob2, lsem, semA, semB):
    wid = _wid()
    rbase = wid * PROWS
    iota = lax.broadcasted_iota(jnp.int32, (L,), 0)

    pltpu.async_copy(fa_hbm.at[pl.ds(rbase, PROWS), :], fab, lsem)
    pltpu.async_copy(so_hbm.at[pl.ds(rbase, PROWS), :], sob, lsem)
    pltpu.async_copy(gf_hbm.at[pl.ds(rbase, PROWS), :], gfb, lsem)
    pltpu.async_copy(fn_hbm.at[pl.ds(rbase, PROWS), :], fnb, lsem)
    pltpu.async_copy(gn_hbm.at[pl.ds(rbase, PROWS), :], gnb, lsem)
    for _ in range(5):
        pltpu.make_async_copy(fa_hbm.at[pl.ds(0, PROWS), :], fab, lsem).wait()

    def start(r, bufs):
        rf, rs, rn, dv, gv, sem = bufs
        pltpu.async_copy(tcell_hbm.at[fab.at[r]], rf, sem)
        pltpu.async_copy(tcell_hbm.at[sob.at[r]], rs, sem)
        pltpu.async_copy(tnet_hbm.at[fnb.at[r]], rn, sem)
        pltpu.async_copy(darr_hbm.at[gfb.at[r]], dv, sem)
        pltpu.async_copy(garr_hbm.at[gnb.at[r]], gv, sem)

    def wait(bufs):
        rf, rs, rn, dv, gv, sem = bufs
        pltpu.make_async_copy(tcell_hbm.at[fab.at[0]], rf, sem).wait()
        pltpu.make_async_copy(tcell_hbm.at[fab.at[0]], rs, sem).wait()
        pltpu.make_async_copy(tnet_hbm.at[fnb.at[0]], rn, sem).wait()
        pltpu.make_async_copy(darr_hbm.at[gfb.at[0]], dv, sem).wait()
        pltpu.make_async_copy(garr_hbm.at[gnb.at[0]], gv, sem).wait()

    bufsA = (rfA, rsA, rnA, dvA, gvA, semA)
    bufsB = (rfB, rsB, rnB, dvB, gvB, semB)

    def tanh16(x):
        e = jnp.exp(2.0 * x)
        return 1.0 - 2.0 / (e + 1.0)

    def compute(r, bufs):
        rf, rs, rn, dv, gv, _ = bufs
        for v in range(8):
            ridx = iota + v * L

            def col(ref, c):
                return plsc.load_gather(ref, [ridx, jnp.full((L,), c,
                                                             jnp.int32)])
            a = col(rf, 0)
            e_ = col(rf, 3)
            sxf = col(rf, 5)
            syf = col(rf, 6)
            b = col(rs, 1)
            f_ = col(rs, 4)
            sxs = col(rs, 5)
            sys_ = col(rs, 6)
            c_ = col(rn, 0)
            h_ = col(rn, 2)
            d_ = dv[pl.ds(v * L, L)]
            g_ = gv[pl.ds(v * L, L)]
            sdis = a + b + c_
            sdef = d_ + e_ + f_ + g_ + h_
            dis = jnp.exp(-2.0 + 15.0 * tanh16(sdis))
            bmin = jnp.minimum((sxf + sxs) * 0.5, (syf + sys_) * 0.5)
            ob1[r, pl.ds(v * L, L)] = dis + bmin
            ob2[r, pl.ds(v * L, L)] = tanh16(sdef) * TWO_PI

    start(0, bufsA)

    def rowpair(i, _):
        r0 = i * 2
        start(r0 + 1, bufsB)
        wait(bufsA)
        compute(r0, bufsA)

        @pl.when(r0 + 2 < PROWS)
        def _():
            start(r0 + 2, bufsA)
        wait(bufsB)
        compute(r0 + 1, bufsB)
        return 0
    lax.fori_loop(0, PROWS // 2, rowpair, 0)

    pltpu.sync_copy(ob1, o1_hbm.at[pl.ds(rbase, PROWS), :])
    pltpu.sync_copy(ob2, o2_hbm.at[pl.ds(rbase, PROWS), :])


_pair_call = pl.kernel(
    _pair_body,
    out_type=[_f32((PPAIR // 128, 128)), _f32((PPAIR // 128, 128))],
    mesh=_mesh,
    compiler_params=pltpu.CompilerParams(needs_layout_passes=False, use_tc_tiling_on_sc=False),
    scratch_types=[
        pltpu.VMEM((PROWS, 128), jnp.int32),
        pltpu.VMEM((PROWS, 128), jnp.int32),
        pltpu.VMEM((PROWS, 128), jnp.int32),
        pltpu.VMEM((PROWS, 128), jnp.int32),
        pltpu.VMEM((PROWS, 128), jnp.int32),
        pltpu.VMEM((128, L), jnp.float32),
        pltpu.VMEM((128, L), jnp.float32),
        pltpu.VMEM((128, L), jnp.float32),
        pltpu.VMEM((128,), jnp.float32),
        pltpu.VMEM((128,), jnp.float32),
        pltpu.VMEM((128, L), jnp.float32),
        pltpu.VMEM((128, L), jnp.float32),
        pltpu.VMEM((128, L), jnp.float32),
        pltpu.VMEM((128,), jnp.float32),
        pltpu.VMEM((128,), jnp.float32),
        pltpu.VMEM((PROWS, 128), jnp.float32),
        pltpu.VMEM((PROWS, 128), jnp.float32),
        pltpu.SemaphoreType.DMA,
        pltpu.SemaphoreType.DMA,
        pltpu.SemaphoreType.DMA,
    ],
)


DB = 512
DGRID = NPAD // DB


def _d1_body(cf, csum, cmax, ccnt, nf, nsum, nmax, ncnt,
             wc, wn, wu, ws, wt, bc, bn, bs_bias, bt_bias,
             u_out, tnet_out, s_out):
    ccnt_ = ccnt[...]
    ncnt_ = ncnt[...]
    cmean = csum[...] / jnp.maximum(ccnt_, 1.0)
    cmx = jnp.where(ccnt_ > 0, cmax[...], 0.0)
    nmean = nsum[...] / jnp.maximum(ncnt_, 1.0)
    nmx = jnp.where(ncnt_ > 0, nmax[...], 0.0)
    wc_ = wc[...]
    wn_ = wn[...]
    hc = jnp.tanh(
        jnp.dot(cf[...], wc_[0:128], preferred_element_type=jnp.float32)
        + jnp.dot(cmean, wc_[128:256], preferred_element_type=jnp.float32)
        + jnp.dot(cmx, wc_[256:384], preferred_element_type=jnp.float32)
        + bc[...])
    hn = jnp.tanh(
        jnp.dot(nf[...], wn_[0:128], preferred_element_type=jnp.float32)
        + jnp.dot(nmean, wn_[128:256], preferred_element_type=jnp.float32)
        + jnp.dot(nmx, wn_[256:384], preferred_element_type=jnp.float32)
        + bn[...])
    u_out[...] = jnp.dot(hn, wu[...], preferred_element_type=jnp.float32)
    tnet_out[...] = (jnp.dot(hn, wt[...], preferred_element_type=jnp.float32)
                     + bt_bias[...])
    s_out[...] = (jnp.dot(hc, ws[...], preferred_element_type=jnp.float32)
                  + bs_bias[...])


def _d1(cf, csum, cmax, ccnt, nf, nsum, nmax, ncnt,
        wc, wn, wu, ws, wt, bc, bn, bs_bias, bt_bias):
    row = pl.BlockSpec((DB, 128), lambda i: (i, 0))
    row1 = pl.BlockSpec((DB, 1), lambda i: (i, 0))
    row16 = pl.BlockSpec((DB, 16), lambda i: (i, 0))
    full = lambda shape: pl.BlockSpec(shape, lambda i: tuple(0 for _ in shape))
    return pl.pallas_call(
        _d1_body,
        grid=(DGRID,),
        in_specs=[row, row, row, row1, row, row, row, row1,
                  full((384, 128)), full((384, 128)), full((128, 16)),
                  full((128, 16)), full((128, 16)), full((1, 128)),
                  full((1, 128)), full((1, 16)), full((1, 16))],
        out_specs=[row16, row16, row16],
        out_shape=[_f32((NPAD, 16)), _f32((NPAD, 16)), _f32((NPAD, 16))],
    )(cf, csum, cmax, ccnt, nf, nsum, nmax, ncnt,
      wc, wn, wu, ws, wt, bc, bn, bs_bias, bt_bias)


PB = 20480
PGRID = EPIN // PB


def _p1_body(pf, wp, bp, we, be, ew_out):
    hp = jnp.tanh(jnp.dot(pf[...], wp[...],
                          preferred_element_type=jnp.float32) + bp[...])
    ew_out[...] = jnp.tanh(jnp.dot(hp, we[...],
                                   preferred_element_type=jnp.float32)
                           + be[...])


def _p1(pf, wp, bp, we, be):
    full = lambda shape: pl.BlockSpec(shape, lambda i: tuple(0 for _ in shape))
    return pl.pallas_call(
        _p1_body,
        grid=(PGRID,),
        in_specs=[pl.BlockSpec((PB, 16), lambda i: (i, 0)),
                  full((16, 16)), full((1, 16)), full((16, 1)), full((1, 1))],
        out_specs=pl.BlockSpec((PB, 1), lambda i: (i, 0)),
        out_shape=_f32((EPIN, 1)),
    )(pf, wp, bp, we, be)


def _d2_body(s_in, w0, w1, cnt, size, tcell_out):
    t = s_in[...] + (w0[...] + w1[...]) / jnp.maximum(cnt[...], 1.0)
    tcell_out[...] = jnp.concatenate(
        [t[:, 0:5], size[...], jnp.zeros((DB, 9), jnp.float32)], axis=1)


def _d2(s_in, w0, w1, cnt, size):
    row16 = pl.BlockSpec((DB, 16), lambda i: (i, 0))
    return pl.pallas_call(
        _d2_body,
        grid=(DGRID,),
        in_specs=[row16, row16, row16, pl.BlockSpec((DB, 1), lambda i: (i, 0)),
                  pl.BlockSpec((DB, 2), lambda i: (i, 0))],
        out_specs=row16,
        out_shape=_f32((NPAD, 16)),
    )(s_in, w0, w1, cnt, size)


def kernel(cell_feat, net_feat, pin_feat, cell_size, pin_cell, pin_net,
           fathers, sons, grandfathers, fs_nets, gf_nets,
           W_cell, b_cell, W_net, b_net, W_pin, b_pin, W_ew, b_ew,
           W_self, W_neigh, b_sage, W_dis, b_dis, W_def, b_def):
    f32 = jnp.float32
    i32 = jnp.int32

    padn = NPAD - NCELL
    cf_p = jnp.concatenate([cell_feat, jnp.zeros((padn, 128), f32)])
    nf_p = jnp.concatenate([net_feat, jnp.zeros((padn, 128), f32)])
    size_p = jnp.concatenate([cell_size, jnp.zeros((padn, 2), f32)])

    padp = EPIN - NPIN
    ar = jnp.arange(padp, dtype=i32)
    pad_dst = NCELL + (ar % padn)
    pc_p = jnp.concatenate([pin_cell.astype(i32), pad_dst])
    pn_p = jnp.concatenate([pin_net.astype(i32), pad_dst])
    pf_p = jnp.concatenate([pin_feat, jnp.zeros((padp, 16), f32)])

    dsrc_net = jnp.stack([pn_p.reshape(NCHUNK, CH),
                          pc_p.reshape(NCHUNK, CH)], axis=1)
    dsrc_cell = jnp.stack([pc_p.reshape(NCHUNK, CH),
                           pn_p.reshape(NCHUNK, CH)], axis=1)

    padq = PPAIR - NPAIR
    arq = jnp.arange(padq, dtype=i32)
    padq_idx = arq % NCELL
    fa2 = jnp.concatenate([fathers.astype(i32), padq_idx]).reshape(-1, 128)
    so2 = jnp.concatenate([sons.astype(i32), padq_idx]).reshape(-1, 128)
    gf2 = jnp.concatenate([grandfathers.astype(i32), padq_idx]).reshape(-1, 128)
    fn2 = jnp.concatenate([fs_nets.astype(i32), padq_idx]).reshape(-1, 128)
    gn2 = jnp.concatenate([gf_nets.astype(i32), padq_idx]).reshape(-1, 128)

    Wd_f, Wd_s, Wd_n = W_dis[0:128], W_dis[128:256], W_dis[256:384]
    We_g, We_f, We_s = W_def[0:128], W_def[128:256], W_def[256:384]
    We_gn, We_fn = W_def[384:512], W_def[512:640]
    cols = [Wd_f, Wd_s, We_g, We_f, We_s]
    WU = jnp.concatenate([W_neigh @ w for w in cols], axis=1)
    WS = jnp.concatenate([W_self @ w for w in cols], axis=1)
    kb = jnp.concatenate([b_sage @ w for w in cols])
    z11 = jnp.zeros((128, 11), f32)
    WU16 = jnp.concatenate([WU, z11], axis=1)
    WS16 = jnp.concatenate([WS, z11], axis=1)
    WT16 = jnp.concatenate([Wd_n, We_gn, We_fn, jnp.zeros((128, 13), f32)],
                           axis=1)
    bs_bias = jnp.concatenate([kb, jnp.zeros((11,), f32)]).reshape(1, 16)
    bt_bias = jnp.concatenate([b_dis, b_def, jnp.zeros((14,), f32)]
                              ).reshape(1, 16)

    nsum_f, nmax_f, ncnt = _seg_call(dsrc_net, cf_p)
    csum_f, cmax_f, ccnt = _seg_call(dsrc_cell, nf_p)
    nsum = nsum_f.reshape(NPAD, 128)
    nmax = nmax_f.reshape(NPAD, 128)
    csum = csum_f.reshape(NPAD, 128)
    cmax = cmax_f.reshape(NPAD, 128)

    U, TNET, S = _d1(cf_p, csum, cmax, ccnt.reshape(NPAD, 1),
                     nf_p, nsum, nmax, ncnt.reshape(NPAD, 1),
                     W_cell, W_net, WU16, WS16, WT16,
                     b_cell.reshape(1, 128), b_net.reshape(1, 128),
                     bs_bias, bt_bias)
    ew = _p1(pf_p, W_pin, b_pin.reshape(1, 16), W_ew, b_ew.reshape(1, 1))
    ew_masked = jnp.where(jnp.arange(EPIN, dtype=i32)[:, None] < NPIN,
                          ew, 0.0)

    wacc = _wseg_call(pn_p.reshape(-1, 128), pc_p.reshape(-1, 128),
                      ew_masked.reshape(-1, 128), U)[0]

    TCELL = _d2(S, wacc[0], wacc[1], ccnt.reshape(NPAD, 1), size_p)
    d_arr = TCELL[:, 2]
    g_arr = TNET[:, 1]

    o1, o2 = _pair_call(fa2, so2, gf2, fn2, gn2, TCELL, TNET, d_arr, g_arr)
    edge_dis = o1.reshape(-1)[:NPAIR]
    edge_deflect = o2.reshape(-1)[:NPAIR]
    return (edge_dis, edge_deflect)


# speedup vs baseline: 1.2223x; 1.2223x over previous
import functools
import math

import jax
import jax.numpy as jnp
from jax import lax
from jax.experimental import pallas as pl
from jax.experimental.pallas import tpu as pltpu
from jax.experimental.pallas import tpu_sc as plsc

NCELL = 50000
NNET = 50000
NPIN = 800000
NPAIR = 400000

NC, NS, L = 2, 16, 16
NW = NC * NS

NPAD = 50176
PASSES = 4
SLICE = 392
EPIN = 819200
CH = 2048
NCHUNK = EPIN // CH
MB = 128
FLUSH_AT = MB - 16
PPAIR = 409600

_mesh = plsc.VectorSubcoreMesh(core_axis_name="c", subcore_axis_name="s")


def _wid():
    return lax.axis_index("s") * NC + lax.axis_index("c")


def _f32(shape):
    return jax.ShapeDtypeStruct(shape, jnp.float32)


def _seg_body(dsrc_hbm, table_hbm, sum_hbm, max_hbm, cnt_hbm,
              sumtbl, maxtbl, cnttbl, dsrcbuf, mrel, msrc, rows,
              sem0, sem1, gsem):
    wid = _wid()
    zero16 = jnp.zeros((L,), jnp.float32)
    ninf16 = jnp.full((L,), -jnp.inf, jnp.float32)
    iota = lax.broadcasted_iota(jnp.int32, (L,), 0)

    def init_msrc(i, _):
        msrc[pl.ds(i * L, L)] = (wid * 251 + i * L + iota) % NCELL
        return 0
    lax.fori_loop(0, MB // L, init_msrc, 0)

    lane0 = iota == 0
    one16 = jnp.ones((L,), jnp.float32)

    def flush(count):
        pltpu.async_copy(table_hbm.at[msrc.at[pl.ds(0, MB)]], rows, gsem).wait()

        def acc(r, _):
            dl = mrel[pl.ds(r, L)][0]
            off = dl * 128
            for j in range(8):
                g = rows[r, pl.ds(j * L, L)]
                sl = pl.ds(off + j * L, L)
                sumtbl[sl] = sumtbl[sl] + g
                maxtbl[sl] = jnp.maximum(maxtbl[sl], g)
            plsc.addupdate_scatter(cnttbl, [jnp.full((L,), dl, jnp.int32)],
                                   one16, mask=lane0)
            return 0
        lax.fori_loop(0, count, acc, 0)

    def do_pass(p, _):
        base = (wid * PASSES + p) * SLICE

        def initrow(i, _):
            sumtbl[pl.ds(i * L, L)] = zero16
            maxtbl[pl.ds(i * L, L)] = ninf16
            return 0
        lax.fori_loop(0, SLICE * 128 // L, initrow, 0)

        def initcnt(i, _):
            cnttbl[pl.ds(i * L, L)] = zero16
            return 0
        lax.fori_loop(0, SLICE // 8 // 2 + 1, initcnt, 0)

        def start_load(c, slot):
            pltpu.async_copy(dsrc_hbm.at[c], dsrcbuf.at[slot],
                             sem0 if slot == 0 else sem1)

        def wait_load(slot):
            pltpu.make_async_copy(dsrc_hbm.at[0], dsrcbuf.at[slot],
                                  sem0 if slot == 0 else sem1).wait()

        def scan_chunk(slot, cursor):
            UNROLL = 4

            def scan_group(g, cur):
                packed = []
                for u in range(UNROLL):
                    v = g * UNROLL + u
                    d = dsrcbuf[slot, 0, pl.ds(v * L, L)]
                    rel = d - base
                    m = (rel >= 0) & (rel < SLICE)
                    s = dsrcbuf[slot, 1, pl.ds(v * L, L)]
                    key = jnp.where(m, rel, jnp.int32(0x7FFFFFFF))
                    sk, sv = plsc.sort_key_val(key, s)
                    n = plsc.all_reduce_population_count(m)
                    packed.append((sk, sv, n))
                for u in range(UNROLL):
                    sk, sv, n = packed[u]
                    mrel[pl.ds(cur, L)] = sk
                    msrc[pl.ds(cur, L)] = sv
                    cur2 = cur + n[0]
                    pred = cur2 >= FLUSH_AT

                    @pl.when(pred)
                    def _():
                        flush(cur2)
                    cur = jnp.where(pred, 0, cur2)
                return cur
            return lax.fori_loop(0, CH // L // UNROLL, scan_group, cursor)

        start_load(0, 0)

        def chunk_pair(i, cursor):
            start_load(2 * i + 1, 1)
            wait_load(0)
            cursor = scan_chunk(0, cursor)

            @pl.when(2 * i + 2 < NCHUNK)
            def _():
                start_load(2 * i + 2, 0)
            wait_load(1)
            cursor = scan_chunk(1, cursor)
            return cursor
        cursor = lax.fori_loop(0, NCHUNK // 2, chunk_pair, 0)

        @pl.when(cursor > 0)
        def _():
            flush(cursor)

        pltpu.sync_copy(sumtbl, sum_hbm.at[pl.ds(base * 128, SLICE * 128)])
        pltpu.sync_copy(maxtbl, max_hbm.at[pl.ds(base * 128, SLICE * 128)])
        pltpu.sync_copy(cnttbl.at[pl.ds(0, SLICE)],
                        cnt_hbm.at[pl.ds(base, SLICE)])
        return 0
    lax.fori_loop(0, PASSES, do_pass, 0)


_seg_call = pl.kernel(
    _seg_body,
    out_type=[_f32((NPAD * 128,)), _f32((NPAD * 128,)), _f32((NPAD,))],
    mesh=_mesh,
    compiler_params=pltpu.CompilerParams(needs_layout_passes=False),
    scratch_types=[
        pltpu.VMEM((SLICE * 128,), jnp.float32),
        pltpu.VMEM((SLICE * 128,), jnp.float32),
        pltpu.VMEM((SLICE + 8,), jnp.float32),
        pltpu.VMEM((2, 2, CH), jnp.int32),
        pltpu.VMEM((MB + L,), jnp.int32),
        pltpu.VMEM((MB + L,), jnp.int32),
        pltpu.VMEM((MB, 128), jnp.float32),
        pltpu.SemaphoreType.DMA,
        pltpu.SemaphoreType.DMA,
        pltpu.SemaphoreType.DMA,
    ],
)


RPT = NPAD // NS
ROWCH = 8
TROWS = EPIN // 128 // NW


def _wseg_body(pn_hbm, pc_hbm, ew_hbm, u_hbm, wacc_hbm,
               nbuf, cbuf, ebuf, urowsA, urowsB, zbuf, shared,
               lsem, gsemA, gsemB):
    wid = _wid()
    sid = lax.axis_index("s")
    cid = lax.axis_index("c")
    zero16 = jnp.zeros((L,), jnp.float32)

    def initz(i, _):
        zbuf[i, :] = zero16
        return 0
    lax.fori_loop(0, RPT // NS, initz, 0)

    def initsh(k, _):
        pltpu.sync_copy(zbuf, shared.at[pl.ds(sid * RPT + k * (RPT // NS),
                                              RPT // NS), :])
        return 0
    lax.fori_loop(0, NS, initsh, 0)
    plsc.subcore_barrier()

    def chunk(ci, _):
        rowbase = wid * TROWS + ci * ROWCH
        pltpu.async_copy(pn_hbm.at[pl.ds(rowbase, ROWCH), :], nbuf, lsem)
        pltpu.async_copy(pc_hbm.at[pl.ds(rowbase, ROWCH), :], cbuf, lsem)
        pltpu.async_copy(ew_hbm.at[pl.ds(rowbase, ROWCH), :], ebuf, lsem)
        for _ in range(3):
            pltpu.make_async_copy(pn_hbm.at[pl.ds(0, ROWCH), :], nbuf,
                                  lsem).wait()

        pltpu.async_copy(u_hbm.at[nbuf.at[0]], urowsA, gsemA)
        for k in range(ROWCH):
            cur, csem = (urowsA, gsemA) if k % 2 == 0 else (urowsB, gsemB)
            nxt, nsem = (urowsB, gsemB) if k % 2 == 0 else (urowsA, gsemA)
            if k < ROWCH - 1:
                pltpu.async_copy(u_hbm.at[nbuf.at[k + 1]], nxt, nsem)
            pltpu.make_async_copy(u_hbm.at[nbuf.at[k]], cur, csem).wait()

            def scale(g, _):
                ev = ebuf[k, pl.ds(g * L, L)]
                for j in range(L):
                    r = g * L + j
                    cur[r, :] = cur[r, :] * ev[j]
                return 0
            lax.fori_loop(0, 128 // L, scale, 0)
            pltpu.sync_copy(cur, shared.at[cbuf.at[k]], add=True)
        return 0
    lax.fori_loop(0, TROWS // ROWCH, chunk, 0)

    plsc.subcore_barrier()
    pltpu.sync_copy(shared.at[pl.ds(sid * RPT, RPT), :],
                    wacc_hbm.at[cid, pl.ds(sid * RPT, RPT), :])


_wseg_call = pl.kernel(
    _wseg_body,
    out_type=[_f32((NC, NPAD, L))],
    mesh=_mesh,
    compiler_params=pltpu.CompilerParams(needs_layout_passes=False, use_tc_tiling_on_sc=False),
    scratch_types=[
        pltpu.VMEM((ROWCH, 128), jnp.int32),
        pltpu.VMEM((ROWCH, 128), jnp.int32),
        pltpu.VMEM((ROWCH, 128), jnp.float32),
        pltpu.VMEM((128, L), jnp.float32),
        pltpu.VMEM((128, L), jnp.float32),
        pltpu.VMEM((RPT // NS, L), jnp.float32),
        pltpu.VMEM_SHARED((NPAD, L), jnp.float32),
        pltpu.SemaphoreType.DMA,
        pltpu.SemaphoreType.DMA,
        pltpu.SemaphoreType.DMA,
    ],
)


PROWS = PPAIR // 128 // NW
TWO_PI = 2.0 * math.pi


def _pair_body(fa_hbm, so_hbm, gf_hbm, fn_hbm, gn_hbm,
               tcell_hbm, tnet_hbm, darr_hbm, garr_hbm,
               o1_hbm, o2_hbm,
               fab, sob, gfb, fnb, gnb,
               rfA, rsA, rnA, dvA, gvA, rfB, rsB, rnB, dvB, gvB,
               ob1, ob2, lsem, semA, semB):
    wid = _wid()
    rbase = wid * PROWS
    iota = lax.broadcasted_iota(jnp.int32, (L,), 0)

    pltpu.async_copy(fa_hbm.at[pl.ds(rbase, PROWS), :], fab, lsem)
    pltpu.async_copy(so_hbm.at[pl.ds(rbase, PROWS), :], sob, lsem)
    pltpu.async_copy(gf_hbm.at[pl.ds(rbase, PROWS), :], gfb, lsem)
    pltpu.async_copy(fn_hbm.at[pl.ds(rbase, PROWS), :], fnb, lsem)
    pltpu.async_copy(gn_hbm.at[pl.ds(rbase, PROWS), :], gnb, lsem)
    for _ in range(5):
        pltpu.make_async_copy(fa_hbm.at[pl.ds(0, PROWS), :], fab, lsem).wait()

    def start(r, bufs):
        rf, rs, rn, dv, gv, sem = bufs
        pltpu.async_copy(tcell_hbm.at[fab.at[r]], rf, sem)
        pltpu.async_copy(tcell_hbm.at[sob.at[r]], rs, sem)
        pltpu.async_copy(tnet_hbm.at[fnb.at[r]], rn, sem)
        pltpu.async_copy(darr_hbm.at[gfb.at[r]], dv, sem)
        pltpu.async_copy(garr_hbm.at[gnb.at[r]], gv, sem)

    def wait(bufs):
        rf, rs, rn, dv, gv, sem = bufs
        pltpu.make_async_copy(tcell_hbm.at[fab.at[0]], rf, sem).wait()
        pltpu.make_async_copy(tcell_hbm.at[fab.at[0]], rs, sem).wait()
        pltpu.make_async_copy(tnet_hbm.at[fnb.at[0]], rn, sem).wait()
        pltpu.make_async_copy(darr_hbm.at[gfb.at[0]], dv, sem).wait()
        pltpu.make_async_copy(garr_hbm.at[gnb.at[0]], gv, sem).wait()

    bufsA = (rfA, rsA, rnA, dvA, gvA, semA)
    bufsB = (rfB, rsB, rnB, dvB, gvB, semB)

    def tanh16(x):
        e = jnp.exp(2.0 * x)
        return 1.0 - 2.0 / (e + 1.0)

    def compute(r, bufs):
        rf, rs, rn, dv, gv, _ = bufs
        for v in range(8):
            ridx = iota + v * L

            def col(ref, c):
                return plsc.load_gather(ref, [ridx, jnp.full((L,), c,
                                                             jnp.int32)])
            a = col(rf, 0)
            e_ = col(rf, 3)
            sxf = col(rf, 5)
            syf = col(rf, 6)
            b = col(rs, 1)
            f_ = col(rs, 4)
            sxs = col(rs, 5)
            sys_ = col(rs, 6)
            c_ = col(rn, 0)
            h_ = col(rn, 2)
            d_ = dv[pl.ds(v * L, L)]
            g_ = gv[pl.ds(v * L, L)]
            sdis = a + b + c_
            sdef = d_ + e_ + f_ + g_ + h_
            dis = jnp.exp(-2.0 + 15.0 * tanh16(sdis))
            bmin = jnp.minimum((sxf + sxs) * 0.5, (syf + sys_) * 0.5)
            ob1[r, pl.ds(v * L, L)] = dis + bmin
            ob2[r, pl.ds(v * L, L)] = tanh16(sdef) * TWO_PI

    start(0, bufsA)

    def rowpair(i, _):
        r0 = i * 2
        start(r0 + 1, bufsB)
        wait(bufsA)
        compute(r0, bufsA)

        @pl.when(r0 + 2 < PROWS)
        def _():
            start(r0 + 2, bufsA)
        wait(bufsB)
        compute(r0 + 1, bufsB)
        return 0
    lax.fori_loop(0, PROWS // 2, rowpair, 0)

    pltpu.sync_copy(ob1, o1_hbm.at[pl.ds(rbase, PROWS), :])
    pltpu.sync_copy(ob2, o2_hbm.at[pl.ds(rbase, PROWS), :])


_pair_call = pl.kernel(
    _pair_body,
    out_type=[_f32((PPAIR // 128, 128)), _f32((PPAIR // 128, 128))],
    mesh=_mesh,
    compiler_params=pltpu.CompilerParams(needs_layout_passes=False, use_tc_tiling_on_sc=False),
    scratch_types=[
        pltpu.VMEM((PROWS, 128), jnp.int32),
        pltpu.VMEM((PROWS, 128), jnp.int32),
        pltpu.VMEM((PROWS, 128), jnp.int32),
        pltpu.VMEM((PROWS, 128), jnp.int32),
        pltpu.VMEM((PROWS, 128), jnp.int32),
        pltpu.VMEM((128, L), jnp.float32),
        pltpu.VMEM((128, L), jnp.float32),
        pltpu.VMEM((128, L), jnp.float32),
        pltpu.VMEM((128,), jnp.float32),
        pltpu.VMEM((128,), jnp.float32),
        pltpu.VMEM((128, L), jnp.float32),
        pltpu.VMEM((128, L), jnp.float32),
        pltpu.VMEM((128, L), jnp.float32),
        pltpu.VMEM((128,), jnp.float32),
        pltpu.VMEM((128,), jnp.float32),
        pltpu.VMEM((PROWS, 128), jnp.float32),
        pltpu.VMEM((PROWS, 128), jnp.float32),
        pltpu.SemaphoreType.DMA,
        pltpu.SemaphoreType.DMA,
        pltpu.SemaphoreType.DMA,
    ],
)


DB = 512
DGRID = NPAD // DB


def _d1_body(cf, csum, cmax, ccnt, nf, nsum, nmax, ncnt,
             wc, wn, wu, ws, wt, bc, bn, bs_bias, bt_bias,
             u_out, tnet_out, s_out):
    ccnt_ = ccnt[...]
    ncnt_ = ncnt[...]
    cmean = csum[...] / jnp.maximum(ccnt_, 1.0)
    cmx = jnp.where(ccnt_ > 0, cmax[...], 0.0)
    nmean = nsum[...] / jnp.maximum(ncnt_, 1.0)
    nmx = jnp.where(ncnt_ > 0, nmax[...], 0.0)
    wc_ = wc[...]
    wn_ = wn[...]
    hc = jnp.tanh(
        jnp.dot(cf[...], wc_[0:128], preferred_element_type=jnp.float32)
        + jnp.dot(cmean, wc_[128:256], preferred_element_type=jnp.float32)
        + jnp.dot(cmx, wc_[256:384], preferred_element_type=jnp.float32)
        + bc[...])
    hn = jnp.tanh(
        jnp.dot(nf[...], wn_[0:128], preferred_element_type=jnp.float32)
        + jnp.dot(nmean, wn_[128:256], preferred_element_type=jnp.float32)
        + jnp.dot(nmx, wn_[256:384], preferred_element_type=jnp.float32)
        + bn[...])
    u_out[...] = jnp.dot(hn, wu[...], preferred_element_type=jnp.float32)
    tnet_out[...] = (jnp.dot(hn, wt[...], preferred_element_type=jnp.float32)
                     + bt_bias[...])
    s_out[...] = (jnp.dot(hc, ws[...], preferred_element_type=jnp.float32)
                  + bs_bias[...])


def _d1(cf, csum, cmax, ccnt, nf, nsum, nmax, ncnt,
        wc, wn, wu, ws, wt, bc, bn, bs_bias, bt_bias):
    row = pl.BlockSpec((DB, 128), lambda i: (i, 0))
    row1 = pl.BlockSpec((DB, 1), lambda i: (i, 0))
    row16 = pl.BlockSpec((DB, 16), lambda i: (i, 0))
    full = lambda shape: pl.BlockSpec(shape, lambda i: tuple(0 for _ in shape))
    return pl.pallas_call(
        _d1_body,
        grid=(DGRID,),
        in_specs=[row, row, row, row1, row, row, row, row1,
                  full((384, 128)), full((384, 128)), full((128, 16)),
                  full((128, 16)), full((128, 16)), full((1, 128)),
                  full((1, 128)), full((1, 16)), full((1, 16))],
        out_specs=[row16, row16, row16],
        out_shape=[_f32((NPAD, 16)), _f32((NPAD, 16)), _f32((NPAD, 16))],
    )(cf, csum, cmax, ccnt, nf, nsum, nmax, ncnt,
      wc, wn, wu, ws, wt, bc, bn, bs_bias, bt_bias)


PB = 20480
PGRID = EPIN // PB


def _p1_body(pf, wp, bp, we, be, ew_out):
    hp = jnp.tanh(jnp.dot(pf[...], wp[...],
                          preferred_element_type=jnp.float32) + bp[...])
    ew_out[...] = jnp.tanh(jnp.dot(hp, we[...],
                                   preferred_element_type=jnp.float32)
                           + be[...])


def _p1(pf, wp, bp, we, be):
    full = lambda shape: pl.BlockSpec(shape, lambda i: tuple(0 for _ in shape))
    return pl.pallas_call(
        _p1_body,
        grid=(PGRID,),
        in_specs=[pl.BlockSpec((PB, 16), lambda i: (i, 0)),
                  full((16, 16)), full((1, 16)), full((16, 1)), full((1, 1))],
        out_specs=pl.BlockSpec((PB, 1), lambda i: (i, 0)),
        out_shape=_f32((EPIN, 1)),
    )(pf, wp, bp, we, be)


def _d2_body(s_in, w0, w1, cnt, size, tcell_out):
    t = s_in[...] + (w0[...] + w1[...]) / jnp.maximum(cnt[...], 1.0)
    tcell_out[...] = jnp.concatenate(
        [t[:, 0:5], size[...], jnp.zeros((DB, 9), jnp.float32)], axis=1)


def _d2(s_in, w0, w1, cnt, size):
    row16 = pl.BlockSpec((DB, 16), lambda i: (i, 0))
    return pl.pallas_call(
        _d2_body,
        grid=(DGRID,),
        in_specs=[row16, row16, row16, pl.BlockSpec((DB, 1), lambda i: (i, 0)),
                  pl.BlockSpec((DB, 2), lambda i: (i, 0))],
        out_specs=row16,
        out_shape=_f32((NPAD, 16)),
    )(s_in, w0, w1, cnt, size)


def kernel(cell_feat, net_feat, pin_feat, cell_size, pin_cell, pin_net,
           fathers, sons, grandfathers, fs_nets, gf_nets,
           W_cell, b_cell, W_net, b_net, W_pin, b_pin, W_ew, b_ew,
           W_self, W_neigh, b_sage, W_dis, b_dis, W_def, b_def):
    f32 = jnp.float32
    i32 = jnp.int32

    padn = NPAD - NCELL
    cf_p = jnp.concatenate([cell_feat, jnp.zeros((padn, 128), f32)])
    nf_p = jnp.concatenate([net_feat, jnp.zeros((padn, 128), f32)])
    size_p = jnp.concatenate([cell_size, jnp.zeros((padn, 2), f32)])

    padp = EPIN - NPIN
    ar = jnp.arange(padp, dtype=i32)
    pad_dst = NCELL + (ar % padn)
    pc_p = jnp.concatenate([pin_cell.astype(i32), pad_dst])
    pn_p = jnp.concatenate([pin_net.astype(i32), pad_dst])
    pf_p = jnp.concatenate([pin_feat, jnp.zeros((padp, 16), f32)])

    dsrc_net = jnp.stack([pn_p.reshape(NCHUNK, CH),
                          pc_p.reshape(NCHUNK, CH)], axis=1)
    dsrc_cell = jnp.stack([pc_p.reshape(NCHUNK, CH),
                           pn_p.reshape(NCHUNK, CH)], axis=1)

    padq = PPAIR - NPAIR
    arq = jnp.arange(padq, dtype=i32)
    padq_idx = arq % NCELL
    fa2 = jnp.concatenate([fathers.astype(i32), padq_idx]).reshape(-1, 128)
    so2 = jnp.concatenate([sons.astype(i32), padq_idx]).reshape(-1, 128)
    gf2 = jnp.concatenate([grandfathers.astype(i32), padq_idx]).reshape(-1, 128)
    fn2 = jnp.concatenate([fs_nets.astype(i32), padq_idx]).reshape(-1, 128)
    gn2 = jnp.concatenate([gf_nets.astype(i32), padq_idx]).reshape(-1, 128)

    Wd_f, Wd_s, Wd_n = W_dis[0:128], W_dis[128:256], W_dis[256:384]
    We_g, We_f, We_s = W_def[0:128], W_def[128:256], W_def[256:384]
    We_gn, We_fn = W_def[384:512], W_def[512:640]
    cols = [Wd_f, Wd_s, We_g, We_f, We_s]
    WU = jnp.concatenate([W_neigh @ w for w in cols], axis=1)
    WS = jnp.concatenate([W_self @ w for w in cols], axis=1)
    kb = jnp.concatenate([b_sage @ w for w in cols])
    z11 = jnp.zeros((128, 11), f32)
    WU16 = jnp.concatenate([WU, z11], axis=1)
    WS16 = jnp.concatenate([WS, z11], axis=1)
    WT16 = jnp.concatenate([Wd_n, We_gn, We_fn, jnp.zeros((128, 13), f32)],
                           axis=1)
    bs_bias = jnp.concatenate([kb, jnp.zeros((11,), f32)]).reshape(1, 16)
    bt_bias = jnp.concatenate([b_dis, b_def, jnp.zeros((14,), f32)]
                              ).reshape(1, 16)

    nsum_f, nmax_f, ncnt = _seg_call(dsrc_net, cf_p)
    csum_f, cmax_f, ccnt = _seg_call(dsrc_cell, nf_p)
    nsum = nsum_f.reshape(NPAD, 128)
    nmax = nmax_f.reshape(NPAD, 128)
    csum = csum_f.reshape(NPAD, 128)
    cmax = cmax_f.reshape(NPAD, 128)

    U, TNET, S = _d1(cf_p, csum, cmax, ccnt.reshape(NPAD, 1),
                     nf_p, nsum, nmax, ncnt.reshape(NPAD, 1),
                     W_cell, W_net, WU16, WS16, WT16,
                     b_cell.reshape(1, 128), b_net.reshape(1, 128),
                     bs_bias, bt_bias)
    ew = _p1(pf_p, W_pin, b_pin.reshape(1, 16), W_ew, b_ew.reshape(1, 1))
    ew_masked = jnp.where(jnp.arange(EPIN, dtype=i32)[:, None] < NPIN,
                          ew, 0.0)

    wacc = _wseg_call(pn_p.reshape(-1, 128), pc_p.reshape(-1, 128),
                      ew_masked.reshape(-1, 128), U)[0]

    TCELL = _d2(S, wacc[0], wacc[1], ccnt.reshape(NPAD, 1), size_p)
    d_arr = TCELL[:, 2]
    g_arr = TNET[:, 1]

    o1, o2 = _pair_call(fa2, so2, gf2, fn2, gn2, TCELL, TNET, d_arr, g_arr)
    edge_dis = o1.reshape(-1)[:NPAIR]
    edge_deflect = o2.reshape(-1)[:NPAIR]
    return (edge_dis, edge_deflect)


# speedup vs baseline: 1.2399x; 1.0144x over previous
import functools
import math

import jax
import jax.numpy as jnp
from jax import lax
from jax.experimental import pallas as pl
from jax.experimental.pallas import tpu as pltpu
from jax.experimental.pallas import tpu_sc as plsc

NCELL = 50000
NNET = 50000
NPIN = 800000
NPAIR = 400000

NC, NS, L = 2, 16, 16
NW = NC * NS

NPAD = 50176
PASSES = 4
SLICE = 392
EPIN = 819200
CH = 1024
NCHUNK = EPIN // CH
MB = 128
FLUSH_AT = MB - 16
PPAIR = 409600

_mesh = plsc.VectorSubcoreMesh(core_axis_name="c", subcore_axis_name="s")


def _wid():
    return lax.axis_index("s") * NC + lax.axis_index("c")


def _f32(shape):
    return jax.ShapeDtypeStruct(shape, jnp.float32)


def _seg_body(dsrc_hbm, table_hbm, sum_hbm, max_hbm, cnt_hbm,
              maxtbl, cnttbl, dsrcbuf, mrel, msrc, midx, rows, shsum,
              sem0, sem1, gsem):
    wid = _wid()
    sid = lax.axis_index("s")
    zero16 = jnp.zeros((L,), jnp.float32)
    ninf16 = jnp.full((L,), -jnp.inf, jnp.float32)
    iota = lax.broadcasted_iota(jnp.int32, (L,), 0)
    lane0 = iota == 0
    one16 = jnp.ones((L,), jnp.float32)

    def init_msrc(i, _):
        msrc[pl.ds(i * L, L)] = (wid * 251 + i * L + iota) % NCELL
        return 0
    lax.fori_loop(0, MB // L, init_msrc, 0)

    def init_midx(i, _):
        midx[0, pl.ds(i * L, L)] = iota * 0 + sid * SLICE
        return 0
    lax.fori_loop(0, MB // L, init_midx, 0)

    def zero_rows_from(count):
        def zrow(r, _):
            for j in range(8):
                rows[r, pl.ds(j * L, L)] = zero16
            return 0
        lax.fori_loop(count, MB, zrow, 0)

    def flush(count):
        pltpu.async_copy(table_hbm.at[msrc], rows, gsem).wait()
        zero_rows_from(count)
        pltpu.sync_copy(rows, shsum.at[midx.at[0]], add=True)

        def acc(i, _):
            r = i * 2
            v = mrel[pl.ds(r, L)]
            dl0 = v[0]
            dl1 = v[1]
            for j in range(8):
                g = rows[r, pl.ds(j * L, L)]
                sl = pl.ds(j * L, L)
                maxtbl[dl0, sl] = jnp.maximum(maxtbl[dl0, sl], g)
            plsc.addupdate_scatter(cnttbl, [jnp.full((L,), dl0, jnp.int32)],
                                   one16, mask=lane0)

            @pl.when(r + 1 < count)
            def _():
                for j in range(8):
                    g = rows[r + 1, pl.ds(j * L, L)]
                    sl = pl.ds(j * L, L)
                    maxtbl[dl1, sl] = jnp.maximum(maxtbl[dl1, sl], g)
                plsc.addupdate_scatter(cnttbl,
                                       [jnp.full((L,), dl1, jnp.int32)],
                                       one16, mask=lane0)
            return 0
        lax.fori_loop(0, (count + 1) // 2, acc, 0)

    def do_pass(p, _):
        base = (p * NW + wid) * SLICE

        def initrow(i, _):
            for j in range(8):
                maxtbl[i, pl.ds(j * L, L)] = ninf16
            return 0
        lax.fori_loop(0, SLICE, initrow, 0)

        def initcnt(i, _):
            cnttbl[pl.ds(i * L, L)] = zero16
            return 0
        lax.fori_loop(0, (SLICE + L) // L, initcnt, 0)

        zero_rows_from(0)
        for k in range(3):
            pltpu.sync_copy(rows,
                            shsum.at[pl.ds(sid * SLICE + k * MB, MB), :])
        pltpu.sync_copy(rows.at[pl.ds(0, SLICE - 3 * MB), :],
                        shsum.at[pl.ds(sid * SLICE + 3 * MB,
                                       SLICE - 3 * MB), :])

        def start_load(c, slot):
            pltpu.async_copy(dsrc_hbm.at[c], dsrcbuf.at[slot],
                             sem0 if slot == 0 else sem1)

        def wait_load(slot):
            pltpu.make_async_copy(dsrc_hbm.at[0], dsrcbuf.at[slot],
                                  sem0 if slot == 0 else sem1).wait()

        def scan_chunk(slot, cursor):
            UNROLL = 4

            def scan_group(g, cur):
                packed = []
                for u in range(UNROLL):
                    v = g * UNROLL + u
                    w = dsrcbuf[slot, pl.ds(v * L, L)]
                    d = lax.shift_right_logical(w, 16)
                    rel = d - base
                    m = (rel >= 0) & (rel < SLICE)
                    s = w & jnp.int32(0xFFFF)
                    key = jnp.where(m, rel, jnp.int32(0x7FFFFFFF))
                    sk, sv = plsc.sort_key_val(key, s)
                    sidx = jnp.minimum(sk, SLICE - 1) + sid * SLICE
                    n = plsc.all_reduce_population_count(m)
                    packed.append((sk, sv, sidx, n))
                for u in range(UNROLL):
                    sk, sv, sidx, n = packed[u]
                    mrel[pl.ds(cur, L)] = sk
                    msrc[pl.ds(cur, L)] = sv
                    midx[0, pl.ds(cur, L)] = sidx
                    cur2 = cur + n[0]
                    pred = cur2 >= FLUSH_AT

                    @pl.when(pred)
                    def _():
                        flush(cur2)
                    cur = jnp.where(pred, 0, cur2)
                return cur
            return lax.fori_loop(0, CH // L // UNROLL, scan_group, cursor)

        start_load(0, 0)

        def chunk_pair(i, cursor):
            start_load(2 * i + 1, 1)
            wait_load(0)
            cursor = scan_chunk(0, cursor)

            @pl.when(2 * i + 2 < NCHUNK)
            def _():
                start_load(2 * i + 2, 0)
            wait_load(1)
            cursor = scan_chunk(1, cursor)
            return cursor
        cursor = lax.fori_loop(0, NCHUNK // 2, chunk_pair, 0)

        @pl.when(cursor > 0)
        def _():
            flush(cursor)

        pltpu.sync_copy(maxtbl, max_hbm.at[pl.ds(base, SLICE), :])
        pltpu.sync_copy(cnttbl.at[pl.ds(0, SLICE)],
                        cnt_hbm.at[pl.ds(base, SLICE)])
        pltpu.sync_copy(shsum.at[pl.ds(sid * SLICE, SLICE), :],
                        sum_hbm.at[pl.ds(base, SLICE), :])
        return 0
    lax.fori_loop(0, PASSES, do_pass, 0)


_seg_call = pl.kernel(
    _seg_body,
    out_type=[_f32((NPAD, 128)), _f32((NPAD, 128)), _f32((NPAD,))],
    mesh=_mesh,
    compiler_params=pltpu.CompilerParams(needs_layout_passes=False),
    scratch_types=[
        pltpu.VMEM((SLICE, 128), jnp.float32),
        pltpu.VMEM((SLICE + L,), jnp.float32),
        pltpu.VMEM((2, CH), jnp.int32),
        pltpu.VMEM((MB + L,), jnp.int32),
        pltpu.VMEM((MB,), jnp.int32),
        pltpu.VMEM((1, MB), jnp.int32),
        pltpu.VMEM((MB, 128), jnp.float32),
        pltpu.VMEM_SHARED((NS * SLICE, 128), jnp.float32),
        pltpu.SemaphoreType.DMA,
        pltpu.SemaphoreType.DMA,
        pltpu.SemaphoreType.DMA,
    ],
)


RPT = NPAD // NS
ROWCH = 8
TROWS = EPIN // 128 // NW


def _wseg_body(pn_hbm, pc_hbm, ew_hbm, u_hbm, wacc_hbm,
               nbuf, cbuf, ebuf, urowsA, urowsB, zbuf, shared,
               lsem, gsemA, gsemB):
    wid = _wid()
    sid = lax.axis_index("s")
    cid = lax.axis_index("c")
    zero16 = jnp.zeros((L,), jnp.float32)

    def initz(i, _):
        zbuf[i, :] = zero16
        return 0
    lax.fori_loop(0, RPT // NS, initz, 0)

    def initsh(k, _):
        pltpu.sync_copy(zbuf, shared.at[pl.ds(sid * RPT + k * (RPT // NS),
                                              RPT // NS), :])
        return 0
    lax.fori_loop(0, NS, initsh, 0)
    plsc.subcore_barrier()

    def chunk(ci, _):
        rowbase = wid * TROWS + ci * ROWCH
        pltpu.async_copy(pn_hbm.at[pl.ds(rowbase, ROWCH), :], nbuf, lsem)
        pltpu.async_copy(pc_hbm.at[pl.ds(rowbase, ROWCH), :], cbuf, lsem)
        pltpu.async_copy(ew_hbm.at[pl.ds(rowbase, ROWCH), :], ebuf, lsem)
        for _ in range(3):
            pltpu.make_async_copy(pn_hbm.at[pl.ds(0, ROWCH), :], nbuf,
                                  lsem).wait()

        pltpu.async_copy(u_hbm.at[nbuf.at[0]], urowsA, gsemA)
        for k in range(ROWCH):
            cur, csem = (urowsA, gsemA) if k % 2 == 0 else (urowsB, gsemB)
            nxt, nsem = (urowsB, gsemB) if k % 2 == 0 else (urowsA, gsemA)
            if k < ROWCH - 1:
                pltpu.async_copy(u_hbm.at[nbuf.at[k + 1]], nxt, nsem)
            pltpu.make_async_copy(u_hbm.at[nbuf.at[k]], cur, csem).wait()

            def scale(g, _):
                ev = ebuf[k, pl.ds(g * L, L)]
                for j in range(L):
                    r = g * L + j
                    cur[r, :] = cur[r, :] * ev[j]
                return 0
            lax.fori_loop(0, 128 // L, scale, 0)
            pltpu.sync_copy(cur, shared.at[cbuf.at[k]], add=True)
        return 0
    lax.fori_loop(0, TROWS // ROWCH, chunk, 0)

    plsc.subcore_barrier()
    pltpu.sync_copy(shared.at[pl.ds(sid * RPT, RPT), :],
                    wacc_hbm.at[cid, pl.ds(sid * RPT, RPT), :])


_wseg_call = pl.kernel(
    _wseg_body,
    out_type=[_f32((NC, NPAD, L))],
    mesh=_mesh,
    compiler_params=pltpu.CompilerParams(needs_layout_passes=False, use_tc_tiling_on_sc=False),
    scratch_types=[
        pltpu.VMEM((ROWCH, 128), jnp.int32),
        pltpu.VMEM((ROWCH, 128), jnp.int32),
        pltpu.VMEM((ROWCH, 128), jnp.float32),
        pltpu.VMEM((128, L), jnp.float32),
        pltpu.VMEM((128, L), jnp.float32),
        pltpu.VMEM((RPT // NS, L), jnp.float32),
        pltpu.VMEM_SHARED((NPAD, L), jnp.float32),
        pltpu.SemaphoreType.DMA,
        pltpu.SemaphoreType.DMA,
        pltpu.SemaphoreType.DMA,
    ],
)


PROWS = PPAIR // 128 // NW
TWO_PI = 2.0 * math.pi


def _pair_body(fa_hbm, so_hbm, gf_hbm, fn_hbm, gn_hbm,
               tcell_hbm, tnet_hbm, darr_hbm, garr_hbm,
               o1_hbm, o2_hbm,
               fab, sob, gfb, fnb, gnb,
               rfA, rsA, rnA, dvA, gvA, rfB, rsB, rnB, dvB, gvB,
               ob1, ob2, lsem, semA, semB):
    wid = _wid()
    rbase = wid * PROWS
    iota = lax.broadcasted_iota(jnp.int32, (L,), 0)

    pltpu.async_copy(fa_hbm.at[pl.ds(rbase, PROWS), :], fab, lsem)
    pltpu.async_copy(so_hbm.at[pl.ds(rbase, PROWS), :], sob, lsem)
    pltpu.async_copy(gf_hbm.at[pl.ds(rbase, PROWS), :], gfb, lsem)
    pltpu.async_copy(fn_hbm.at[pl.ds(rbase, PROWS), :], fnb, lsem)
    pltpu.async_copy(gn_hbm.at[pl.ds(rbase, PROWS), :], gnb, lsem)
    for _ in range(5):
        pltpu.make_async_copy(fa_hbm.at[pl.ds(0, PROWS), :], fab, lsem).wait()

    def start(r, bufs):
        rf, rs, rn, dv, gv, sem = bufs
        pltpu.async_copy(tcell_hbm.at[fab.at[r]], rf, sem)
        pltpu.async_copy(tcell_hbm.at[sob.at[r]], rs, sem)
        pltpu.async_copy(tnet_hbm.at[fnb.at[r]], rn, sem)
        pltpu.async_copy(darr_hbm.at[gfb.at[r]], dv, sem)
        pltpu.async_copy(garr_hbm.at[gnb.at[r]], gv, sem)

    def wait(bufs):
        rf, rs, rn, dv, gv, sem = bufs
        pltpu.make_async_copy(tcell_hbm.at[fab.at[0]], rf, sem).wait()
        pltpu.make_async_copy(tcell_hbm.at[fab.at[0]], rs, sem).wait()
        pltpu.make_async_copy(tnet_hbm.at[fnb.at[0]], rn, sem).wait()
        pltpu.make_async_copy(darr_hbm.at[gfb.at[0]], dv, sem).wait()
        pltpu.make_async_copy(garr_hbm.at[gnb.at[0]], gv, sem).wait()

    bufsA = (rfA, rsA, rnA, dvA, gvA, semA)
    bufsB = (rfB, rsB, rnB, dvB, gvB, semB)

    def tanh16(x):
        e = jnp.exp(2.0 * x)
        return 1.0 - 2.0 / (e + 1.0)

    def compute(r, bufs):
        rf, rs, rn, dv, gv, _ = bufs
        for v in range(8):
            ridx = iota + v * L

            def col(ref, c):
                return plsc.load_gather(ref, [ridx, jnp.full((L,), c,
                                                             jnp.int32)])
            a = col(rf, 0)
            e_ = col(rf, 3)
            sxf = col(rf, 5)
            syf = col(rf, 6)
            b = col(rs, 1)
            f_ = col(rs, 4)
            sxs = col(rs, 5)
            sys_ = col(rs, 6)
            c_ = col(rn, 0)
            h_ = col(rn, 2)
            d_ = dv[pl.ds(v * L, L)]
            g_ = gv[pl.ds(v * L, L)]
            sdis = a + b + c_
            sdef = d_ + e_ + f_ + g_ + h_
            dis = jnp.exp(-2.0 + 15.0 * tanh16(sdis))
            bmin = jnp.minimum((sxf + sxs) * 0.5, (syf + sys_) * 0.5)
            ob1[r, pl.ds(v * L, L)] = dis + bmin
            ob2[r, pl.ds(v * L, L)] = tanh16(sdef) * TWO_PI

    start(0, bufsA)

    def rowpair(i, _):
        r0 = i * 2
        start(r0 + 1, bufsB)
        wait(bufsA)
        compute(r0, bufsA)

        @pl.when(r0 + 2 < PROWS)
        def _():
            start(r0 + 2, bufsA)
        wait(bufsB)
        compute(r0 + 1, bufsB)
        return 0
    lax.fori_loop(0, PROWS // 2, rowpair, 0)

    pltpu.sync_copy(ob1, o1_hbm.at[pl.ds(rbase, PROWS), :])
    pltpu.sync_copy(ob2, o2_hbm.at[pl.ds(rbase, PROWS), :])


_pair_call = pl.kernel(
    _pair_body,
    out_type=[_f32((PPAIR // 128, 128)), _f32((PPAIR // 128, 128))],
    mesh=_mesh,
    compiler_params=pltpu.CompilerParams(needs_layout_passes=False, use_tc_tiling_on_sc=False),
    scratch_types=[
        pltpu.VMEM((PROWS, 128), jnp.int32),
        pltpu.VMEM((PROWS, 128), jnp.int32),
        pltpu.VMEM((PROWS, 128), jnp.int32),
        pltpu.VMEM((PROWS, 128), jnp.int32),
        pltpu.VMEM((PROWS, 128), jnp.int32),
        pltpu.VMEM((128, L), jnp.float32),
        pltpu.VMEM((128, L), jnp.float32),
        pltpu.VMEM((128, L), jnp.float32),
        pltpu.VMEM((128,), jnp.float32),
        pltpu.VMEM((128,), jnp.float32),
        pltpu.VMEM((128, L), jnp.float32),
        pltpu.VMEM((128, L), jnp.float32),
        pltpu.VMEM((128, L), jnp.float32),
        pltpu.VMEM((128,), jnp.float32),
        pltpu.VMEM((128,), jnp.float32),
        pltpu.VMEM((PROWS, 128), jnp.float32),
        pltpu.VMEM((PROWS, 128), jnp.float32),
        pltpu.SemaphoreType.DMA,
        pltpu.SemaphoreType.DMA,
        pltpu.SemaphoreType.DMA,
    ],
)


DB = 512
DGRID = NPAD // DB


def _d1_body(cf, csum, cmax, ccnt, nf, nsum, nmax, ncnt,
             wc, wn, wu, ws, wt, bc, bn, bs_bias, bt_bias,
             u_out, tnet_out, s_out):
    ccnt_ = ccnt[...]
    ncnt_ = ncnt[...]
    cmean = csum[...] / jnp.maximum(ccnt_, 1.0)
    cmx = jnp.where(ccnt_ > 0, cmax[...], 0.0)
    nmean = nsum[...] / jnp.maximum(ncnt_, 1.0)
    nmx = jnp.where(ncnt_ > 0, nmax[...], 0.0)
    wc_ = wc[...]
    wn_ = wn[...]
    hc = jnp.tanh(
        jnp.dot(cf[...], wc_[0:128], preferred_element_type=jnp.float32)
        + jnp.dot(cmean, wc_[128:256], preferred_element_type=jnp.float32)
        + jnp.dot(cmx, wc_[256:384], preferred_element_type=jnp.float32)
        + bc[...])
    hn = jnp.tanh(
        jnp.dot(nf[...], wn_[0:128], preferred_element_type=jnp.float32)
        + jnp.dot(nmean, wn_[128:256], preferred_element_type=jnp.float32)
        + jnp.dot(nmx, wn_[256:384], preferred_element_type=jnp.float32)
        + bn[...])
    u_out[...] = jnp.dot(hn, wu[...], preferred_element_type=jnp.float32)
    tnet_out[...] = (jnp.dot(hn, wt[...], preferred_element_type=jnp.float32)
                     + bt_bias[...])
    s_out[...] = (jnp.dot(hc, ws[...], preferred_element_type=jnp.float32)
                  + bs_bias[...])


def _d1(cf, csum, cmax, ccnt, nf, nsum, nmax, ncnt,
        wc, wn, wu, ws, wt, bc, bn, bs_bias, bt_bias):
    row = pl.BlockSpec((DB, 128), lambda i: (i, 0))
    row1 = pl.BlockSpec((DB, 1), lambda i: (i, 0))
    row16 = pl.BlockSpec((DB, 16), lambda i: (i, 0))
    full = lambda shape: pl.BlockSpec(shape, lambda i: tuple(0 for _ in shape))
    return pl.pallas_call(
        _d1_body,
        grid=(DGRID,),
        in_specs=[row, row, row, row1, row, row, row, row1,
                  full((384, 128)), full((384, 128)), full((128, 16)),
                  full((128, 16)), full((128, 16)), full((1, 128)),
                  full((1, 128)), full((1, 16)), full((1, 16))],
        out_specs=[row16, row16, row16],
        out_shape=[_f32((NPAD, 16)), _f32((NPAD, 16)), _f32((NPAD, 16))],
    )(cf, csum, cmax, ccnt, nf, nsum, nmax, ncnt,
      wc, wn, wu, ws, wt, bc, bn, bs_bias, bt_bias)


PB = 20480
PGRID = EPIN // PB


def _p1_body(pf, wp, bp, we, be, ew_out):
    hp = jnp.tanh(jnp.dot(pf[...], wp[...],
                          preferred_element_type=jnp.float32) + bp[...])
    ew_out[...] = jnp.tanh(jnp.dot(hp, we[...],
                                   preferred_element_type=jnp.float32)
                           + be[...])


def _p1(pf, wp, bp, we, be):
    full = lambda shape: pl.BlockSpec(shape, lambda i: tuple(0 for _ in shape))
    return pl.pallas_call(
        _p1_body,
        grid=(PGRID,),
        in_specs=[pl.BlockSpec((PB, 16), lambda i: (i, 0)),
                  full((16, 16)), full((1, 16)), full((16, 1)), full((1, 1))],
        out_specs=pl.BlockSpec((PB, 1), lambda i: (i, 0)),
        out_shape=_f32((EPIN, 1)),
    )(pf, wp, bp, we, be)


def _d2_body(s_in, w0, w1, cnt, size, tcell_out):
    t = s_in[...] + (w0[...] + w1[...]) / jnp.maximum(cnt[...], 1.0)
    tcell_out[...] = jnp.concatenate(
        [t[:, 0:5], size[...], jnp.zeros((DB, 9), jnp.float32)], axis=1)


def _d2(s_in, w0, w1, cnt, size):
    row16 = pl.BlockSpec((DB, 16), lambda i: (i, 0))
    return pl.pallas_call(
        _d2_body,
        grid=(DGRID,),
        in_specs=[row16, row16, row16, pl.BlockSpec((DB, 1), lambda i: (i, 0)),
                  pl.BlockSpec((DB, 2), lambda i: (i, 0))],
        out_specs=row16,
        out_shape=_f32((NPAD, 16)),
    )(s_in, w0, w1, cnt, size)


def kernel(cell_feat, net_feat, pin_feat, cell_size, pin_cell, pin_net,
           fathers, sons, grandfathers, fs_nets, gf_nets,
           W_cell, b_cell, W_net, b_net, W_pin, b_pin, W_ew, b_ew,
           W_self, W_neigh, b_sage, W_dis, b_dis, W_def, b_def):
    f32 = jnp.float32
    i32 = jnp.int32

    padn = NPAD - NCELL
    cf_p = jnp.concatenate([cell_feat, jnp.zeros((padn, 128), f32)])
    nf_p = jnp.concatenate([net_feat, jnp.zeros((padn, 128), f32)])
    size_p = jnp.concatenate([cell_size, jnp.zeros((padn, 2), f32)])

    padp = EPIN - NPIN
    ar = jnp.arange(padp, dtype=i32)
    pad_dst = NCELL + (ar % padn)
    pc_p = jnp.concatenate([pin_cell.astype(i32), pad_dst])
    pn_p = jnp.concatenate([pin_net.astype(i32), pad_dst])
    pf_p = jnp.concatenate([pin_feat, jnp.zeros((padp, 16), f32)])

    pn_u = pn_p.astype(jnp.uint32)
    pc_u = pc_p.astype(jnp.uint32)
    dsrc_net = lax.bitcast_convert_type(
        (pn_u << 16) | pc_u, jnp.int32).reshape(NCHUNK, CH)
    dsrc_cell = lax.bitcast_convert_type(
        (pc_u << 16) | pn_u, jnp.int32).reshape(NCHUNK, CH)

    padq = PPAIR - NPAIR
    arq = jnp.arange(padq, dtype=i32)
    padq_idx = arq % NCELL
    fa2 = jnp.concatenate([fathers.astype(i32), padq_idx]).reshape(-1, 128)
    so2 = jnp.concatenate([sons.astype(i32), padq_idx]).reshape(-1, 128)
    gf2 = jnp.concatenate([grandfathers.astype(i32), padq_idx]).reshape(-1, 128)
    fn2 = jnp.concatenate([fs_nets.astype(i32), padq_idx]).reshape(-1, 128)
    gn2 = jnp.concatenate([gf_nets.astype(i32), padq_idx]).reshape(-1, 128)

    Wd_f, Wd_s, Wd_n = W_dis[0:128], W_dis[128:256], W_dis[256:384]
    We_g, We_f, We_s = W_def[0:128], W_def[128:256], W_def[256:384]
    We_gn, We_fn = W_def[384:512], W_def[512:640]
    cols = [Wd_f, Wd_s, We_g, We_f, We_s]
    WU = jnp.concatenate([W_neigh @ w for w in cols], axis=1)
    WS = jnp.concatenate([W_self @ w for w in cols], axis=1)
    kb = jnp.concatenate([b_sage @ w for w in cols])
    z11 = jnp.zeros((128, 11), f32)
    WU16 = jnp.concatenate([WU, z11], axis=1)
    WS16 = jnp.concatenate([WS, z11], axis=1)
    WT16 = jnp.concatenate([Wd_n, We_gn, We_fn, jnp.zeros((128, 13), f32)],
                           axis=1)
    bs_bias = jnp.concatenate([kb, jnp.zeros((11,), f32)]).reshape(1, 16)
    bt_bias = jnp.concatenate([b_dis, b_def, jnp.zeros((14,), f32)]
                              ).reshape(1, 16)

    nsum, nmax, ncnt = _seg_call(dsrc_net, cf_p)
    csum, cmax, ccnt = _seg_call(dsrc_cell, nf_p)

    U, TNET, S = _d1(cf_p, csum, cmax, ccnt.reshape(NPAD, 1),
                     nf_p, nsum, nmax, ncnt.reshape(NPAD, 1),
                     W_cell, W_net, WU16, WS16, WT16,
                     b_cell.reshape(1, 128), b_net.reshape(1, 128),
                     bs_bias, bt_bias)
    ew = _p1(pf_p, W_pin, b_pin.reshape(1, 16), W_ew, b_ew.reshape(1, 1))
    ew_masked = jnp.where(jnp.arange(EPIN, dtype=i32)[:, None] < NPIN,
                          ew, 0.0)

    wacc = _wseg_call(pn_p.reshape(-1, 128), pc_p.reshape(-1, 128),
                      ew_masked.reshape(-1, 128), U)[0]

    TCELL = _d2(S, wacc[0], wacc[1], ccnt.reshape(NPAD, 1), size_p)
    d_arr = TCELL[:, 2]
    g_arr = TNET[:, 1]

    o1, o2 = _pair_call(fa2, so2, gf2, fn2, gn2, TCELL, TNET, d_arr, g_arr)
    edge_dis = o1.reshape(-1)[:NPAIR]
    edge_deflect = o2.reshape(-1)[:NPAIR]
    return (edge_dis, edge_deflect)


# speedup vs baseline: 1.3158x; 1.0613x over previous
import functools
import math

import jax
import jax.numpy as jnp
from jax import lax
from jax.experimental import pallas as pl
from jax.experimental.pallas import tpu as pltpu
from jax.experimental.pallas import tpu_sc as plsc

NCELL = 50000
NNET = 50000
NPIN = 800000
NPAIR = 400000

NC, NS, L = 2, 16, 16
NW = NC * NS

NPAD = 51200
PASSES = 5
SLICE = 320
EPIN = 819200
CH = 1024
NCHUNK = EPIN // CH
MB = 128
PS = 144
FLUSH_AT = MB - 16
PPAIR = 409600

_mesh = plsc.VectorSubcoreMesh(core_axis_name="c", subcore_axis_name="s")


def _wid():
    return lax.axis_index("s") * NC + lax.axis_index("c")


def _f32(shape):
    return jax.ShapeDtypeStruct(shape, jnp.float32)


def _seg_body(dsrc_hbm, table_hbm, sum_hbm, max_hbm, cnt_hbm,
              maxtbl, cnttbl, dsrcbuf, mrel, msrc, midx, rows, shsum,
              sem0, sem1, gsem, ssem):
    wid = _wid()
    sid = lax.axis_index("s")
    zero16 = jnp.zeros((L,), jnp.float32)
    ninf16 = jnp.full((L,), -jnp.inf, jnp.float32)
    iota = lax.broadcasted_iota(jnp.int32, (L,), 0)
    lane0 = iota == 0
    one16 = jnp.ones((L,), jnp.float32)

    def init_msrc(i, _):
        msrc[pl.ds(i * L, L)] = (wid * 251 + i * L + iota) % NCELL
        return 0
    lax.fori_loop(0, 2 * MB // L, init_msrc, 0)

    def init_midx(i, _):
        midx[0, pl.ds(i * L, L)] = iota * 0 + sid * SLICE
        midx[1, pl.ds(i * L, L)] = iota * 0 + sid * SLICE
        return 0
    lax.fori_loop(0, MB // L, init_midx, 0)

    def zero_rows_from(count, hp):
        def zrow(r, _):
            for j in range(8):
                rows[hp * MB + r, pl.ds(j * L, L)] = zero16
            return 0
        lax.fori_loop(count, MB, zrow, 0)

    def rows_half(hp):
        return rows.at[pl.ds(hp * MB, MB), :]

    def msrc_half(hp):
        return msrc.at[pl.ds(hp * MB, MB)]

    def start_gather(hp):
        pltpu.async_copy(table_hbm.at[msrc_half(hp)], rows_half(hp), gsem)

    def wait_gather(hp):
        pltpu.make_async_copy(table_hbm.at[msrc_half(0)], rows_half(hp),
                              gsem).wait()

    def process_batch(count, hp):
        zero_rows_from(count, hp)
        sdesc = pltpu.async_copy(rows_half(hp), shsum.at[midx.at[hp]],
                                 ssem, add=True)

        def acc(i, _):
            r = i * 2
            v = mrel[pl.ds(hp * PS + r, L)]
            dl0 = v[0]
            dl1 = v[1]
            for j in range(8):
                g = rows[hp * MB + r, pl.ds(j * L, L)]
                sl = pl.ds(j * L, L)
                maxtbl[dl0, sl] = jnp.maximum(maxtbl[dl0, sl], g)
            plsc.addupdate_scatter(cnttbl, [jnp.full((L,), dl0, jnp.int32)],
                                   one16, mask=lane0)

            @pl.when(r + 1 < count)
            def _():
                for j in range(8):
                    g = rows[hp * MB + r + 1, pl.ds(j * L, L)]
                    sl = pl.ds(j * L, L)
                    maxtbl[dl1, sl] = jnp.maximum(maxtbl[dl1, sl], g)
                plsc.addupdate_scatter(cnttbl,
                                       [jnp.full((L,), dl1, jnp.int32)],
                                       one16, mask=lane0)
            return 0
        lax.fori_loop(0, (count + 1) // 2, acc, 0)
        sdesc.wait()

    def on_full(count, par, pend):
        opar = 1 - par

        @pl.when(pend > 0)
        def _():
            wait_gather(opar)
            process_batch(pend, opar)
        start_gather(par)

    def do_pass(p, _):
        base = (p * NW + wid) * SLICE

        def initrow(i, _):
            for j in range(8):
                maxtbl[i, pl.ds(j * L, L)] = ninf16
            return 0
        lax.fori_loop(0, SLICE, initrow, 0)

        def initcnt(i, _):
            cnttbl[pl.ds(i * L, L)] = zero16
            return 0
        lax.fori_loop(0, (SLICE + L) // L, initcnt, 0)

        zero_rows_from(0, 0)
        for k in range(2):
            pltpu.sync_copy(rows_half(0),
                            shsum.at[pl.ds(sid * SLICE + k * MB, MB), :])
        pltpu.sync_copy(rows.at[pl.ds(0, SLICE - 2 * MB), :],
                        shsum.at[pl.ds(sid * SLICE + 2 * MB,
                                       SLICE - 2 * MB), :])

        def start_load(c, slot):
            pltpu.async_copy(dsrc_hbm.at[c], dsrcbuf.at[slot],
                             sem0 if slot == 0 else sem1)

        def wait_load(slot):
            pltpu.make_async_copy(dsrc_hbm.at[0], dsrcbuf.at[slot],
                                  sem0 if slot == 0 else sem1).wait()

        def scan_chunk(slot, carry):
            UNROLL = 4

            def scan_group(g, carry):
                cur, par, pend = carry
                packed = []
                for u in range(UNROLL):
                    v = g * UNROLL + u
                    w = dsrcbuf[slot, pl.ds(v * L, L)]
                    d = lax.shift_right_logical(w, 16)
                    rel = d - base
                    m = (rel >= 0) & (rel < SLICE)
                    s = w & jnp.int32(0xFFFF)
                    key = jnp.where(m, rel, jnp.int32(0x7FFFFFFF))
                    sk, sv = plsc.sort_key_val(key, s)
                    sidx = jnp.minimum(sk, SLICE - 1) + sid * SLICE
                    n = plsc.all_reduce_population_count(m)
                    packed.append((sk, sv, sidx, n))
                for u in range(UNROLL):
                    sk, sv, sidx, n = packed[u]
                    mrel[pl.ds(par * PS + cur, L)] = sk
                    msrc[pl.ds(par * MB + cur, L)] = sv
                    midx[par, pl.ds(cur, L)] = sidx
                    cur2 = cur + n[0]
                    pred = cur2 >= FLUSH_AT

                    @pl.when(pred)
                    def _():
                        on_full(cur2, par, pend)
                    par2 = jnp.where(pred, 1 - par, par)
                    pend = jnp.where(pred, cur2, pend)
                    cur = jnp.where(pred, 0, cur2)
                    par = par2
                return (cur, par, pend)
            return lax.fori_loop(0, CH // L // UNROLL, scan_group, carry)

        start_load(0, 0)

        def chunk_pair(i, carry):
            start_load(2 * i + 1, 1)
            wait_load(0)
            carry = scan_chunk(0, carry)

            @pl.when(2 * i + 2 < NCHUNK)
            def _():
                start_load(2 * i + 2, 0)
            wait_load(1)
            carry = scan_chunk(1, carry)
            return carry
        cursor, par, pend = lax.fori_loop(0, NCHUNK // 2, chunk_pair,
                                          (0, 0, 0))

        @pl.when(pend > 0)
        def _():
            wait_gather(1 - par)
            process_batch(pend, 1 - par)

        @pl.when(cursor > 0)
        def _():
            start_gather(par)
            wait_gather(par)
            process_batch(cursor, par)

        pltpu.sync_copy(maxtbl, max_hbm.at[pl.ds(base, SLICE), :])
        pltpu.sync_copy(cnttbl.at[pl.ds(0, SLICE)],
                        cnt_hbm.at[pl.ds(base, SLICE)])
        pltpu.sync_copy(shsum.at[pl.ds(sid * SLICE, SLICE), :],
                        sum_hbm.at[pl.ds(base, SLICE), :])
        return 0
    lax.fori_loop(0, PASSES, do_pass, 0)


_seg_call = pl.kernel(
    _seg_body,
    out_type=[_f32((NPAD, 128)), _f32((NPAD, 128)), _f32((NPAD,))],
    mesh=_mesh,
    compiler_params=pltpu.CompilerParams(needs_layout_passes=False),
    scratch_types=[
        pltpu.VMEM((SLICE, 128), jnp.float32),
        pltpu.VMEM((SLICE + L,), jnp.float32),
        pltpu.VMEM((2, CH), jnp.int32),
        pltpu.VMEM((2 * PS,), jnp.int32),
        pltpu.VMEM((2 * MB,), jnp.int32),
        pltpu.VMEM((2, MB), jnp.int32),
        pltpu.VMEM((2 * MB, 128), jnp.float32),
        pltpu.VMEM_SHARED((NS * SLICE, 128), jnp.float32),
        pltpu.SemaphoreType.DMA,
        pltpu.SemaphoreType.DMA,
        pltpu.SemaphoreType.DMA,
        pltpu.SemaphoreType.DMA,
    ],
)


RPT = NPAD // NS
ROWCH = 8
TROWS = EPIN // 128 // NW


def _wseg_body(pn_hbm, pc_hbm, ew_hbm, u_hbm, wacc_hbm,
               nbuf, cbuf, ebuf, urowsA, urowsB, zbuf, shared,
               lsem, gsemA, gsemB):
    wid = _wid()
    sid = lax.axis_index("s")
    cid = lax.axis_index("c")
    zero16 = jnp.zeros((L,), jnp.float32)

    def initz(i, _):
        zbuf[i, :] = zero16
        return 0
    lax.fori_loop(0, RPT // NS, initz, 0)

    def initsh(k, _):
        pltpu.sync_copy(zbuf, shared.at[pl.ds(sid * RPT + k * (RPT // NS),
                                              RPT // NS), :])
        return 0
    lax.fori_loop(0, NS, initsh, 0)
    plsc.subcore_barrier()

    def chunk(ci, _):
        rowbase = wid * TROWS + ci * ROWCH
        pltpu.async_copy(pn_hbm.at[pl.ds(rowbase, ROWCH), :], nbuf, lsem)
        pltpu.async_copy(pc_hbm.at[pl.ds(rowbase, ROWCH), :], cbuf, lsem)
        pltpu.async_copy(ew_hbm.at[pl.ds(rowbase, ROWCH), :], ebuf, lsem)
        for _ in range(3):
            pltpu.make_async_copy(pn_hbm.at[pl.ds(0, ROWCH), :], nbuf,
                                  lsem).wait()

        pltpu.async_copy(u_hbm.at[nbuf.at[0]], urowsA, gsemA)
        for k in range(ROWCH):
            cur, csem = (urowsA, gsemA) if k % 2 == 0 else (urowsB, gsemB)
            nxt, nsem = (urowsB, gsemB) if k % 2 == 0 else (urowsA, gsemA)
            if k < ROWCH - 1:
                pltpu.async_copy(u_hbm.at[nbuf.at[k + 1]], nxt, nsem)
            pltpu.make_async_copy(u_hbm.at[nbuf.at[k]], cur, csem).wait()

            def scale(g, _):
                ev = ebuf[k, pl.ds(g * L, L)]
                for j in range(L):
                    r = g * L + j
                    cur[r, :] = cur[r, :] * ev[j]
                return 0
            lax.fori_loop(0, 128 // L, scale, 0)
            pltpu.sync_copy(cur, shared.at[cbuf.at[k]], add=True)
        return 0
    lax.fori_loop(0, TROWS // ROWCH, chunk, 0)

    plsc.subcore_barrier()
    pltpu.sync_copy(shared.at[pl.ds(sid * RPT, RPT), :],
                    wacc_hbm.at[cid, pl.ds(sid * RPT, RPT), :])


_wseg_call = pl.kernel(
    _wseg_body,
    out_type=[_f32((NC, NPAD, L))],
    mesh=_mesh,
    compiler_params=pltpu.CompilerParams(needs_layout_passes=False, use_tc_tiling_on_sc=False),
    scratch_types=[
        pltpu.VMEM((ROWCH, 128), jnp.int32),
        pltpu.VMEM((ROWCH, 128), jnp.int32),
        pltpu.VMEM((ROWCH, 128), jnp.float32),
        pltpu.VMEM((128, L), jnp.float32),
        pltpu.VMEM((128, L), jnp.float32),
        pltpu.VMEM((RPT // NS, L), jnp.float32),
        pltpu.VMEM_SHARED((NPAD, L), jnp.float32),
        pltpu.SemaphoreType.DMA,
        pltpu.SemaphoreType.DMA,
        pltpu.SemaphoreType.DMA,
    ],
)


PROWS = PPAIR // 128 // NW
TWO_PI = 2.0 * math.pi


def _pair_body(fa_hbm, so_hbm, gf_hbm, fn_hbm, gn_hbm,
               tcell_hbm, tnet_hbm, darr_hbm, garr_hbm,
               o1_hbm, o2_hbm,
               fab, sob, gfb, fnb, gnb,
               rfA, rsA, rnA, dvA, gvA, rfB, rsB, rnB, dvB, gvB,
               ob1, ob2, lsem, semA, semB):
    wid = _wid()
    rbase = wid * PROWS
    iota = lax.broadcasted_iota(jnp.int32, (L,), 0)

    pltpu.async_copy(fa_hbm.at[pl.ds(rbase, PROWS), :], fab, lsem)
    pltpu.async_copy(so_hbm.at[pl.ds(rbase, PROWS), :], sob, lsem)
    pltpu.async_copy(gf_hbm.at[pl.ds(rbase, PROWS), :], gfb, lsem)
    pltpu.async_copy(fn_hbm.at[pl.ds(rbase, PROWS), :], fnb, lsem)
    pltpu.async_copy(gn_hbm.at[pl.ds(rbase, PROWS), :], gnb, lsem)
    for _ in range(5):
        pltpu.make_async_copy(fa_hbm.at[pl.ds(0, PROWS), :], fab, lsem).wait()

    def start(r, bufs):
        rf, rs, rn, dv, gv, sem = bufs
        pltpu.async_copy(tcell_hbm.at[fab.at[r]], rf, sem)
        pltpu.async_copy(tcell_hbm.at[sob.at[r]], rs, sem)
        pltpu.async_copy(tnet_hbm.at[fnb.at[r]], rn, sem)
        pltpu.async_copy(darr_hbm.at[gfb.at[r]], dv, sem)
        pltpu.async_copy(garr_hbm.at[gnb.at[r]], gv, sem)

    def wait(bufs):
        rf, rs, rn, dv, gv, sem = bufs
        pltpu.make_async_copy(tcell_hbm.at[fab.at[0]], rf, sem).wait()
        pltpu.make_async_copy(tcell_hbm.at[fab.at[0]], rs, sem).wait()
        pltpu.make_async_copy(tnet_hbm.at[fnb.at[0]], rn, sem).wait()
        pltpu.make_async_copy(darr_hbm.at[gfb.at[0]], dv, sem).wait()
        pltpu.make_async_copy(garr_hbm.at[gnb.at[0]], gv, sem).wait()

    bufsA = (rfA, rsA, rnA, dvA, gvA, semA)
    bufsB = (rfB, rsB, rnB, dvB, gvB, semB)

    def tanh16(x):
        e = jnp.exp(2.0 * x)
        return 1.0 - 2.0 / (e + 1.0)

    def compute(r, bufs):
        rf, rs, rn, dv, gv, _ = bufs
        for v in range(8):
            ridx = iota + v * L

            def col(ref, c):
                return plsc.load_gather(ref, [ridx, jnp.full((L,), c,
                                                             jnp.int32)])
            a = col(rf, 0)
            e_ = col(rf, 3)
            sxf = col(rf, 5)
            syf = col(rf, 6)
            b = col(rs, 1)
            f_ = col(rs, 4)
            sxs = col(rs, 5)
            sys_ = col(rs, 6)
            c_ = col(rn, 0)
            h_ = col(rn, 2)
            d_ = dv[pl.ds(v * L, L)]
            g_ = gv[pl.ds(v * L, L)]
            sdis = a + b + c_
            sdef = d_ + e_ + f_ + g_ + h_
            dis = jnp.exp(-2.0 + 15.0 * tanh16(sdis))
            bmin = jnp.minimum((sxf + sxs) * 0.5, (syf + sys_) * 0.5)
            ob1[r, pl.ds(v * L, L)] = dis + bmin
            ob2[r, pl.ds(v * L, L)] = tanh16(sdef) * TWO_PI

    start(0, bufsA)

    def rowpair(i, _):
        r0 = i * 2
        start(r0 + 1, bufsB)
        wait(bufsA)
        compute(r0, bufsA)

        @pl.when(r0 + 2 < PROWS)
        def _():
            start(r0 + 2, bufsA)
        wait(bufsB)
        compute(r0 + 1, bufsB)
        return 0
    lax.fori_loop(0, PROWS // 2, rowpair, 0)

    pltpu.sync_copy(ob1, o1_hbm.at[pl.ds(rbase, PROWS), :])
    pltpu.sync_copy(ob2, o2_hbm.at[pl.ds(rbase, PROWS), :])


_pair_call = pl.kernel(
    _pair_body,
    out_type=[_f32((PPAIR // 128, 128)), _f32((PPAIR // 128, 128))],
    mesh=_mesh,
    compiler_params=pltpu.CompilerParams(needs_layout_passes=False, use_tc_tiling_on_sc=False),
    scratch_types=[
        pltpu.VMEM((PROWS, 128), jnp.int32),
        pltpu.VMEM((PROWS, 128), jnp.int32),
        pltpu.VMEM((PROWS, 128), jnp.int32),
        pltpu.VMEM((PROWS, 128), jnp.int32),
        pltpu.VMEM((PROWS, 128), jnp.int32),
        pltpu.VMEM((128, L), jnp.float32),
        pltpu.VMEM((128, L), jnp.float32),
        pltpu.VMEM((128, L), jnp.float32),
        pltpu.VMEM((128,), jnp.float32),
        pltpu.VMEM((128,), jnp.float32),
        pltpu.VMEM((128, L), jnp.float32),
        pltpu.VMEM((128, L), jnp.float32),
        pltpu.VMEM((128, L), jnp.float32),
        pltpu.VMEM((128,), jnp.float32),
        pltpu.VMEM((128,), jnp.float32),
        pltpu.VMEM((PROWS, 128), jnp.float32),
        pltpu.VMEM((PROWS, 128), jnp.float32),
        pltpu.SemaphoreType.DMA,
        pltpu.SemaphoreType.DMA,
        pltpu.SemaphoreType.DMA,
    ],
)


DB = 512
DGRID = NPAD // DB


def _d1_body(cf, csum, cmax, ccnt, nf, nsum, nmax, ncnt,
             wc, wn, wu, ws, wt, bc, bn, bs_bias, bt_bias,
             u_out, tnet_out, s_out):
    ccnt_ = ccnt[...]
    ncnt_ = ncnt[...]
    cmean = csum[...] / jnp.maximum(ccnt_, 1.0)
    cmx = jnp.where(ccnt_ > 0, cmax[...], 0.0)
    nmean = nsum[...] / jnp.maximum(ncnt_, 1.0)
    nmx = jnp.where(ncnt_ > 0, nmax[...], 0.0)
    wc_ = wc[...]
    wn_ = wn[...]
    hc = jnp.tanh(
        jnp.dot(cf[...], wc_[0:128], preferred_element_type=jnp.float32)
        + jnp.dot(cmean, wc_[128:256], preferred_element_type=jnp.float32)
        + jnp.dot(cmx, wc_[256:384], preferred_element_type=jnp.float32)
        + bc[...])
    hn = jnp.tanh(
        jnp.dot(nf[...], wn_[0:128], preferred_element_type=jnp.float32)
        + jnp.dot(nmean, wn_[128:256], preferred_element_type=jnp.float32)
        + jnp.dot(nmx, wn_[256:384], preferred_element_type=jnp.float32)
        + bn[...])
    u_out[...] = jnp.dot(hn, wu[...], preferred_element_type=jnp.float32)
    tnet_out[...] = (jnp.dot(hn, wt[...], preferred_element_type=jnp.float32)
                     + bt_bias[...])
    s_out[...] = (jnp.dot(hc, ws[...], preferred_element_type=jnp.float32)
                  + bs_bias[...])


def _d1(cf, csum, cmax, ccnt, nf, nsum, nmax, ncnt,
        wc, wn, wu, ws, wt, bc, bn, bs_bias, bt_bias):
    row = pl.BlockSpec((DB, 128), lambda i: (i, 0))
    row1 = pl.BlockSpec((DB, 1), lambda i: (i, 0))
    row16 = pl.BlockSpec((DB, 16), lambda i: (i, 0))
    full = lambda shape: pl.BlockSpec(shape, lambda i: tuple(0 for _ in shape))
    return pl.pallas_call(
        _d1_body,
        grid=(DGRID,),
        in_specs=[row, row, row, row1, row, row, row, row1,
                  full((384, 128)), full((384, 128)), full((128, 16)),
                  full((128, 16)), full((128, 16)), full((1, 128)),
                  full((1, 128)), full((1, 16)), full((1, 16))],
        out_specs=[row16, row16, row16],
        out_shape=[_f32((NPAD, 16)), _f32((NPAD, 16)), _f32((NPAD, 16))],
    )(cf, csum, cmax, ccnt, nf, nsum, nmax, ncnt,
      wc, wn, wu, ws, wt, bc, bn, bs_bias, bt_bias)


PB = 20480
PGRID = EPIN // PB


def _p1_body(pf, wp, bp, we, be, ew_out):
    hp = jnp.tanh(jnp.dot(pf[...], wp[...],
                          preferred_element_type=jnp.float32) + bp[...])
    ew_out[...] = jnp.tanh(jnp.dot(hp, we[...],
                                   preferred_element_type=jnp.float32)
                           + be[...])


def _p1(pf, wp, bp, we, be):
    full = lambda shape: pl.BlockSpec(shape, lambda i: tuple(0 for _ in shape))
    return pl.pallas_call(
        _p1_body,
        grid=(PGRID,),
        in_specs=[pl.BlockSpec((PB, 16), lambda i: (i, 0)),
                  full((16, 16)), full((1, 16)), full((16, 1)), full((1, 1))],
        out_specs=pl.BlockSpec((PB, 1), lambda i: (i, 0)),
        out_shape=_f32((EPIN, 1)),
    )(pf, wp, bp, we, be)


def _d2_body(s_in, w0, w1, cnt, size, tcell_out):
    t = s_in[...] + (w0[...] + w1[...]) / jnp.maximum(cnt[...], 1.0)
    tcell_out[...] = jnp.concatenate(
        [t[:, 0:5], size[...], jnp.zeros((DB, 9), jnp.float32)], axis=1)


def _d2(s_in, w0, w1, cnt, size):
    row16 = pl.BlockSpec((DB, 16), lambda i: (i, 0))
    return pl.pallas_call(
        _d2_body,
        grid=(DGRID,),
        in_specs=[row16, row16, row16, pl.BlockSpec((DB, 1), lambda i: (i, 0)),
                  pl.BlockSpec((DB, 2), lambda i: (i, 0))],
        out_specs=row16,
        out_shape=_f32((NPAD, 16)),
    )(s_in, w0, w1, cnt, size)


def kernel(cell_feat, net_feat, pin_feat, cell_size, pin_cell, pin_net,
           fathers, sons, grandfathers, fs_nets, gf_nets,
           W_cell, b_cell, W_net, b_net, W_pin, b_pin, W_ew, b_ew,
           W_self, W_neigh, b_sage, W_dis, b_dis, W_def, b_def):
    f32 = jnp.float32
    i32 = jnp.int32

    padn = NPAD - NCELL
    cf_p = jnp.concatenate([cell_feat, jnp.zeros((padn, 128), f32)])
    nf_p = jnp.concatenate([net_feat, jnp.zeros((padn, 128), f32)])
    size_p = jnp.concatenate([cell_size, jnp.zeros((padn, 2), f32)])

    padp = EPIN - NPIN
    ar = jnp.arange(padp, dtype=i32)
    pad_dst = NCELL + (ar % padn)
    pc_p = jnp.concatenate([pin_cell.astype(i32), pad_dst])
    pn_p = jnp.concatenate([pin_net.astype(i32), pad_dst])
    pf_p = jnp.concatenate([pin_feat, jnp.zeros((padp, 16), f32)])

    pn_u = pn_p.astype(jnp.uint32)
    pc_u = pc_p.astype(jnp.uint32)
    dsrc_net = lax.bitcast_convert_type(
        (pn_u << 16) | pc_u, jnp.int32).reshape(NCHUNK, CH)
    dsrc_cell = lax.bitcast_convert_type(
        (pc_u << 16) | pn_u, jnp.int32).reshape(NCHUNK, CH)

    padq = PPAIR - NPAIR
    arq = jnp.arange(padq, dtype=i32)
    padq_idx = arq % NCELL
    fa2 = jnp.concatenate([fathers.astype(i32), padq_idx]).reshape(-1, 128)
    so2 = jnp.concatenate([sons.astype(i32), padq_idx]).reshape(-1, 128)
    gf2 = jnp.concatenate([grandfathers.astype(i32), padq_idx]).reshape(-1, 128)
    fn2 = jnp.concatenate([fs_nets.astype(i32), padq_idx]).reshape(-1, 128)
    gn2 = jnp.concatenate([gf_nets.astype(i32), padq_idx]).reshape(-1, 128)

    Wd_f, Wd_s, Wd_n = W_dis[0:128], W_dis[128:256], W_dis[256:384]
    We_g, We_f, We_s = W_def[0:128], W_def[128:256], W_def[256:384]
    We_gn, We_fn = W_def[384:512], W_def[512:640]
    cols = [Wd_f, Wd_s, We_g, We_f, We_s]
    WU = jnp.concatenate([W_neigh @ w for w in cols], axis=1)
    WS = jnp.concatenate([W_self @ w for w in cols], axis=1)
    kb = jnp.concatenate([b_sage @ w for w in cols])
    z11 = jnp.zeros((128, 11), f32)
    WU16 = jnp.concatenate([WU, z11], axis=1)
    WS16 = jnp.concatenate([WS, z11], axis=1)
    WT16 = jnp.concatenate([Wd_n, We_gn, We_fn, jnp.zeros((128, 13), f32)],
                           axis=1)
    bs_bias = jnp.concatenate([kb, jnp.zeros((11,), f32)]).reshape(1, 16)
    bt_bias = jnp.concatenate([b_dis, b_def, jnp.zeros((14,), f32)]
                              ).reshape(1, 16)

    nsum, nmax, ncnt = _seg_call(dsrc_net, cf_p)
    csum, cmax, ccnt = _seg_call(dsrc_cell, nf_p)

    U, TNET, S = _d1(cf_p, csum, cmax, ccnt.reshape(NPAD, 1),
                     nf_p, nsum, nmax, ncnt.reshape(NPAD, 1),
                     W_cell, W_net, WU16, WS16, WT16,
                     b_cell.reshape(1, 128), b_net.reshape(1, 128),
                     bs_bias, bt_bias)
    ew = _p1(pf_p, W_pin, b_pin.reshape(1, 16), W_ew, b_ew.reshape(1, 1))
    ew_masked = jnp.where(jnp.arange(EPIN, dtype=i32)[:, None] < NPIN,
                          ew, 0.0)

    wacc = _wseg_call(pn_p.reshape(-1, 128), pc_p.reshape(-1, 128),
                      ew_masked.reshape(-1, 128), U)[0]

    TCELL = _d2(S, wacc[0], wacc[1], ccnt.reshape(NPAD, 1), size_p)
    d_arr = TCELL[:, 2]
    g_arr = TNET[:, 1]

    o1, o2 = _pair_call(fa2, so2, gf2, fn2, gn2, TCELL, TNET, d_arr, g_arr)
    edge_dis = o1.reshape(-1)[:NPAIR]
    edge_deflect = o2.reshape(-1)[:NPAIR]
    return (edge_dis, edge_deflect)


# speedup vs baseline: 1.9665x; 1.4945x over previous
import functools
import math

import jax
import jax.numpy as jnp
from jax import lax
from jax.experimental import pallas as pl
from jax.experimental.pallas import tpu as pltpu
from jax.experimental.pallas import tpu_sc as plsc

NCELL = 50000
NNET = 50000
NPIN = 800000
NPAIR = 400000

NC, NS, L = 2, 16, 16
NW = NC * NS

NPAD = 51200
PASSES = 5
SLICE = 320
EPIN = 819200
CH = 2048
NCHUNK = EPIN // CH
MB = 128
PS = 144
FLUSH_AT = MB - 32
PPAIR = 409600

_mesh = plsc.VectorSubcoreMesh(core_axis_name="c", subcore_axis_name="s")


def _wid():
    return lax.axis_index("s") * NC + lax.axis_index("c")


def _f32(shape):
    return jax.ShapeDtypeStruct(shape, jnp.float32)


def _seg_body(dsrc_hbm, table_hbm, sum_hbm, max_hbm, cnt_hbm,
              maxtbl, cnttbl, dsrcbuf, mrel, msrc, midx, rows, shsum,
              sem0, sem1, gsem, ssem):
    wid = _wid()
    sid = lax.axis_index("s")
    zero16 = jnp.zeros((L,), jnp.float32)
    ninf16 = jnp.full((L,), -jnp.inf, jnp.float32)
    iota = lax.broadcasted_iota(jnp.int32, (L,), 0)
    lane0 = iota == 0
    one16 = jnp.ones((L,), jnp.float32)

    def init_msrc(i, _):
        msrc[pl.ds(i * L, L)] = (wid * 251 + i * L + iota) % NCELL
        return 0
    lax.fori_loop(0, 2 * MB // L, init_msrc, 0)

    def init_midx(i, _):
        midx[0, pl.ds(i * L, L)] = iota * 0 + sid * SLICE
        midx[1, pl.ds(i * L, L)] = iota * 0 + sid * SLICE
        return 0
    lax.fori_loop(0, MB // L, init_midx, 0)

    def zero_rows_from(count, hp):
        def zrow(r, _):
            for j in range(8):
                rows[hp * MB + r, pl.ds(j * L, L)] = zero16
            return 0
        lax.fori_loop(count, MB, zrow, 0)

    def rows_half(hp):
        return rows.at[pl.ds(hp * MB, MB), :]

    def msrc_half(hp):
        return msrc.at[pl.ds(hp * MB, MB)]

    def start_gather(hp):
        pltpu.async_copy(table_hbm.at[msrc_half(hp)], rows_half(hp), gsem)

    def wait_gather(hp):
        pltpu.make_async_copy(table_hbm.at[msrc_half(0)], rows_half(hp),
                              gsem).wait()

    def process_batch(count, hp):
        zero_rows_from(count, hp)
        sdesc = pltpu.async_copy(rows_half(hp), shsum.at[midx.at[hp]],
                                 ssem, add=True)

        def acc(i, _):
            r = i * 2
            v = mrel[pl.ds(hp * PS + r, L)]
            dl0 = v[0]
            dl1 = v[1]
            for j in range(8):
                g = rows[hp * MB + r, pl.ds(j * L, L)]
                sl = pl.ds(j * L, L)
                maxtbl[dl0, sl] = jnp.maximum(maxtbl[dl0, sl], g)
            plsc.addupdate_scatter(cnttbl, [jnp.full((L,), dl0, jnp.int32)],
                                   one16, mask=lane0)

            @pl.when(r + 1 < count)
            def _():
                for j in range(8):
                    g = rows[hp * MB + r + 1, pl.ds(j * L, L)]
                    sl = pl.ds(j * L, L)
                    maxtbl[dl1, sl] = jnp.maximum(maxtbl[dl1, sl], g)
                plsc.addupdate_scatter(cnttbl,
                                       [jnp.full((L,), dl1, jnp.int32)],
                                       one16, mask=lane0)
            return 0
        lax.fori_loop(0, (count + 1) // 2, acc, 0)
        sdesc.wait()

    def on_full(count, par, pend):
        opar = 1 - par

        @pl.when(pend > 0)
        def _():
            wait_gather(opar)
            process_batch(pend, opar)
        start_gather(par)

    def do_pass(p, _):
        base = (p * NW + wid) * SLICE

        def initrow(i, _):
            for j in range(8):
                maxtbl[i, pl.ds(j * L, L)] = ninf16
            return 0
        lax.fori_loop(0, SLICE, initrow, 0)

        def initcnt(i, _):
            cnttbl[pl.ds(i * L, L)] = zero16
            return 0
        lax.fori_loop(0, (SLICE + L) // L, initcnt, 0)

        zero_rows_from(0, 0)
        for k in range(2):
            pltpu.sync_copy(rows_half(0),
                            shsum.at[pl.ds(sid * SLICE + k * MB, MB), :])
        pltpu.sync_copy(rows.at[pl.ds(0, SLICE - 2 * MB), :],
                        shsum.at[pl.ds(sid * SLICE + 2 * MB,
                                       SLICE - 2 * MB), :])

        def start_load(c, slot):
            pltpu.async_copy(dsrc_hbm.at[c], dsrcbuf.at[slot],
                             sem0 if slot == 0 else sem1)

        def wait_load(slot):
            pltpu.make_async_copy(dsrc_hbm.at[0], dsrcbuf.at[slot],
                                  sem0 if slot == 0 else sem1).wait()

        def scan_chunk(slot, carry):
            UNROLL = 8

            def scan_group(g, carry):
                cur, par, pend = carry
                packed = []
                for u in range(UNROLL):
                    v = g * UNROLL + u
                    w = dsrcbuf[slot, pl.ds(v * L, L)]
                    d = lax.shift_right_logical(w, 16)
                    rel = d - base
                    m = (rel >= 0) & (rel < SLICE)
                    s = w & jnp.int32(0xFFFF)
                    key = jnp.where(m, rel, jnp.int32(0x7FFFFFFF))
                    sk, sv = plsc.sort_key_val(key, s)
                    sidx = jnp.minimum(sk, SLICE - 1) + sid * SLICE
                    n = plsc.all_reduce_population_count(m)
                    packed.append((sk, sv, sidx, n))
                for u in range(UNROLL):
                    sk, sv, sidx, n = packed[u]
                    mrel[pl.ds(par * PS + cur, L)] = sk
                    msrc[pl.ds(par * MB + cur, L)] = sv
                    midx[par, pl.ds(cur, L)] = sidx
                    cur = cur + n[0]
                    if u % 2 == 1:
                        pred = cur >= FLUSH_AT

                        @pl.when(pred)
                        def _():
                            on_full(cur, par, pend)
                        par2 = jnp.where(pred, 1 - par, par)
                        pend = jnp.where(pred, cur, pend)
                        cur = jnp.where(pred, 0, cur)
                        par = par2
                return (cur, par, pend)
            return lax.fori_loop(0, CH // L // UNROLL, scan_group, carry)

        start_load(0, 0)

        def chunk_pair(i, carry):
            start_load(2 * i + 1, 1)
            wait_load(0)
            carry = scan_chunk(0, carry)

            @pl.when(2 * i + 2 < NCHUNK)
            def _():
                start_load(2 * i + 2, 0)
            wait_load(1)
            carry = scan_chunk(1, carry)
            return carry
        cursor, par, pend = lax.fori_loop(0, NCHUNK // 2, chunk_pair,
                                          (0, 0, 0))

        @pl.when(pend > 0)
        def _():
            wait_gather(1 - par)
            process_batch(pend, 1 - par)

        @pl.when(cursor > 0)
        def _():
            start_gather(par)
            wait_gather(par)
            process_batch(cursor, par)

        pltpu.sync_copy(maxtbl, max_hbm.at[pl.ds(base, SLICE), :])
        pltpu.sync_copy(cnttbl.at[pl.ds(0, SLICE)],
                        cnt_hbm.at[pl.ds(base, SLICE)])
        pltpu.sync_copy(shsum.at[pl.ds(sid * SLICE, SLICE), :],
                        sum_hbm.at[pl.ds(base, SLICE), :])
        return 0
    lax.fori_loop(0, PASSES, do_pass, 0)


_seg_call = pl.kernel(
    _seg_body,
    out_type=[_f32((NPAD, 128)), _f32((NPAD, 128)), _f32((NPAD,))],
    mesh=_mesh,
    compiler_params=pltpu.CompilerParams(needs_layout_passes=False),
    scratch_types=[
        pltpu.VMEM((SLICE, 128), jnp.float32),
        pltpu.VMEM((SLICE + L,), jnp.float32),
        pltpu.VMEM((2, CH), jnp.int32),
        pltpu.VMEM((2 * PS,), jnp.int32),
        pltpu.VMEM((2 * MB,), jnp.int32),
        pltpu.VMEM((2, MB), jnp.int32),
        pltpu.VMEM((2 * MB, 128), jnp.float32),
        pltpu.VMEM_SHARED((NS * SLICE, 128), jnp.float32),
        pltpu.SemaphoreType.DMA,
        pltpu.SemaphoreType.DMA,
        pltpu.SemaphoreType.DMA,
        pltpu.SemaphoreType.DMA,
    ],
)


RPT = NPAD // NS
ROWCH = 8
TROWS = EPIN // 128 // NW


def _wseg_body(pn_hbm, pc_hbm, ew_hbm, u_hbm, wacc_hbm,
               nbuf, cbuf, ebuf, urowsA, urowsB, zbuf, shared,
               lsem, gsemA, gsemB):
    wid = _wid()
    sid = lax.axis_index("s")
    cid = lax.axis_index("c")
    zero16 = jnp.zeros((L,), jnp.float32)

    def initz(i, _):
        zbuf[i, :] = zero16
        return 0
    lax.fori_loop(0, RPT // NS, initz, 0)

    def initsh(k, _):
        pltpu.sync_copy(zbuf, shared.at[pl.ds(sid * RPT + k * (RPT // NS),
                                              RPT // NS), :])
        return 0
    lax.fori_loop(0, NS, initsh, 0)
    plsc.subcore_barrier()

    def chunk(ci, _):
        rowbase = wid * TROWS + ci * ROWCH
        pltpu.async_copy(pn_hbm.at[pl.ds(rowbase, ROWCH), :], nbuf, lsem)
        pltpu.async_copy(pc_hbm.at[pl.ds(rowbase, ROWCH), :], cbuf, lsem)
        pltpu.async_copy(ew_hbm.at[pl.ds(rowbase, ROWCH), :], ebuf, lsem)
        for _ in range(3):
            pltpu.make_async_copy(pn_hbm.at[pl.ds(0, ROWCH), :], nbuf,
                                  lsem).wait()

        pltpu.async_copy(u_hbm.at[nbuf.at[0]], urowsA, gsemA)
        for k in range(ROWCH):
            cur, csem = (urowsA, gsemA) if k % 2 == 0 else (urowsB, gsemB)
            nxt, nsem = (urowsB, gsemB) if k % 2 == 0 else (urowsA, gsemA)
            if k < ROWCH - 1:
                pltpu.async_copy(u_hbm.at[nbuf.at[k + 1]], nxt, nsem)
            pltpu.make_async_copy(u_hbm.at[nbuf.at[k]], cur, csem).wait()

            def scale(g, _):
                ev = ebuf[k, pl.ds(g * L, L)]
                for j in range(L):
                    r = g * L + j
                    cur[r, :] = cur[r, :] * ev[j]
                return 0
            lax.fori_loop(0, 128 // L, scale, 0)
            pltpu.sync_copy(cur, shared.at[cbuf.at[k]], add=True)
        return 0
    lax.fori_loop(0, TROWS // ROWCH, chunk, 0)

    plsc.subcore_barrier()
    pltpu.sync_copy(shared.at[pl.ds(sid * RPT, RPT), :],
                    wacc_hbm.at[cid, pl.ds(sid * RPT, RPT), :])


_wseg_call = pl.kernel(
    _wseg_body,
    out_type=[_f32((NC, NPAD, L))],
    mesh=_mesh,
    compiler_params=pltpu.CompilerParams(needs_layout_passes=False, use_tc_tiling_on_sc=False),
    scratch_types=[
        pltpu.VMEM((ROWCH, 128), jnp.int32),
        pltpu.VMEM((ROWCH, 128), jnp.int32),
        pltpu.VMEM((ROWCH, 128), jnp.float32),
        pltpu.VMEM((128, L), jnp.float32),
        pltpu.VMEM((128, L), jnp.float32),
        pltpu.VMEM((RPT // NS, L), jnp.float32),
        pltpu.VMEM_SHARED((NPAD, L), jnp.float32),
        pltpu.SemaphoreType.DMA,
        pltpu.SemaphoreType.DMA,
        pltpu.SemaphoreType.DMA,
    ],
)


PROWS = PPAIR // 128 // NW
TWO_PI = 2.0 * math.pi


def _pair_body(fa_hbm, so_hbm, gf_hbm, fn_hbm, gn_hbm,
               tcell_hbm, tnet_hbm, darr_hbm, garr_hbm,
               o1_hbm, o2_hbm,
               fab, sob, gfb, fnb, gnb,
               rfA, rsA, rnA, dvA, gvA, rfB, rsB, rnB, dvB, gvB,
               ob1, ob2, lsem, semA, semB):
    wid = _wid()
    rbase = wid * PROWS
    iota = lax.broadcasted_iota(jnp.int32, (L,), 0)

    pltpu.async_copy(fa_hbm.at[pl.ds(rbase, PROWS), :], fab, lsem)
    pltpu.async_copy(so_hbm.at[pl.ds(rbase, PROWS), :], sob, lsem)
    pltpu.async_copy(gf_hbm.at[pl.ds(rbase, PROWS), :], gfb, lsem)
    pltpu.async_copy(fn_hbm.at[pl.ds(rbase, PROWS), :], fnb, lsem)
    pltpu.async_copy(gn_hbm.at[pl.ds(rbase, PROWS), :], gnb, lsem)
    for _ in range(5):
        pltpu.make_async_copy(fa_hbm.at[pl.ds(0, PROWS), :], fab, lsem).wait()

    def start(r, bufs):
        rf, rs, rn, dv, gv, sem = bufs
        pltpu.async_copy(tcell_hbm.at[fab.at[r]], rf, sem)
        pltpu.async_copy(tcell_hbm.at[sob.at[r]], rs, sem)
        pltpu.async_copy(tnet_hbm.at[fnb.at[r]], rn, sem)
        pltpu.async_copy(darr_hbm.at[gfb.at[r]], dv, sem)
        pltpu.async_copy(garr_hbm.at[gnb.at[r]], gv, sem)

    def wait(bufs):
        rf, rs, rn, dv, gv, sem = bufs
        pltpu.make_async_copy(tcell_hbm.at[fab.at[0]], rf, sem).wait()
        pltpu.make_async_copy(tcell_hbm.at[fab.at[0]], rs, sem).wait()
        pltpu.make_async_copy(tnet_hbm.at[fnb.at[0]], rn, sem).wait()
        pltpu.make_async_copy(darr_hbm.at[gfb.at[0]], dv, sem).wait()
        pltpu.make_async_copy(garr_hbm.at[gnb.at[0]], gv, sem).wait()

    bufsA = (rfA, rsA, rnA, dvA, gvA, semA)
    bufsB = (rfB, rsB, rnB, dvB, gvB, semB)

    def tanh16(x):
        e = jnp.exp(2.0 * x)
        return 1.0 - 2.0 / (e + 1.0)

    def compute(r, bufs):
        rf, rs, rn, dv, gv, _ = bufs
        for v in range(8):
            ridx = iota + v * L

            def col(ref, c):
                return plsc.load_gather(ref, [ridx, jnp.full((L,), c,
                                                             jnp.int32)])
            a = col(rf, 0)
            e_ = col(rf, 3)
            sxf = col(rf, 5)
            syf = col(rf, 6)
            b = col(rs, 1)
            f_ = col(rs, 4)
            sxs = col(rs, 5)
            sys_ = col(rs, 6)
            c_ = col(rn, 0)
            h_ = col(rn, 2)
            d_ = dv[pl.ds(v * L, L)]
            g_ = gv[pl.ds(v * L, L)]
            sdis = a + b + c_
            sdef = d_ + e_ + f_ + g_ + h_
            dis = jnp.exp(-2.0 + 15.0 * tanh16(sdis))
            bmin = jnp.minimum((sxf + sxs) * 0.5, (syf + sys_) * 0.5)
            ob1[r, pl.ds(v * L, L)] = dis + bmin
            ob2[r, pl.ds(v * L, L)] = tanh16(sdef) * TWO_PI

    start(0, bufsA)

    def rowpair(i, _):
        r0 = i * 2
        start(r0 + 1, bufsB)
        wait(bufsA)
        compute(r0, bufsA)

        @pl.when(r0 + 2 < PROWS)
        def _():
            start(r0 + 2, bufsA)
        wait(bufsB)
        compute(r0 + 1, bufsB)
        return 0
    lax.fori_loop(0, PROWS // 2, rowpair, 0)

    pltpu.sync_copy(ob1, o1_hbm.at[pl.ds(rbase, PROWS), :])
    pltpu.sync_copy(ob2, o2_hbm.at[pl.ds(rbase, PROWS), :])


_pair_call = pl.kernel(
    _pair_body,
    out_type=[_f32((PPAIR // 128, 128)), _f32((PPAIR // 128, 128))],
    mesh=_mesh,
    compiler_params=pltpu.CompilerParams(needs_layout_passes=False, use_tc_tiling_on_sc=False),
    scratch_types=[
        pltpu.VMEM((PROWS, 128), jnp.int32),
        pltpu.VMEM((PROWS, 128), jnp.int32),
        pltpu.VMEM((PROWS, 128), jnp.int32),
        pltpu.VMEM((PROWS, 128), jnp.int32),
        pltpu.VMEM((PROWS, 128), jnp.int32),
        pltpu.VMEM((128, L), jnp.float32),
        pltpu.VMEM((128, L), jnp.float32),
        pltpu.VMEM((128, L), jnp.float32),
        pltpu.VMEM((128,), jnp.float32),
        pltpu.VMEM((128,), jnp.float32),
        pltpu.VMEM((128, L), jnp.float32),
        pltpu.VMEM((128, L), jnp.float32),
        pltpu.VMEM((128, L), jnp.float32),
        pltpu.VMEM((128,), jnp.float32),
        pltpu.VMEM((128,), jnp.float32),
        pltpu.VMEM((PROWS, 128), jnp.float32),
        pltpu.VMEM((PROWS, 128), jnp.float32),
        pltpu.SemaphoreType.DMA,
        pltpu.SemaphoreType.DMA,
        pltpu.SemaphoreType.DMA,
    ],
)


DB = 512
DGRID = NPAD // DB


def _d1_body(cf, csum, cmax, ccnt, nf, nsum, nmax, ncnt,
             wc, wn, wu, ws, wt, bc, bn, bs_bias, bt_bias,
             u_out, tnet_out, s_out):
    ccnt_ = ccnt[...]
    ncnt_ = ncnt[...]
    cmean = csum[...] / jnp.maximum(ccnt_, 1.0)
    cmx = jnp.where(ccnt_ > 0, cmax[...], 0.0)
    nmean = nsum[...] / jnp.maximum(ncnt_, 1.0)
    nmx = jnp.where(ncnt_ > 0, nmax[...], 0.0)
    wc_ = wc[...]
    wn_ = wn[...]
    hc = jnp.tanh(
        jnp.dot(cf[...], wc_[0:128], preferred_element_type=jnp.float32)
        + jnp.dot(cmean, wc_[128:256], preferred_element_type=jnp.float32)
        + jnp.dot(cmx, wc_[256:384], preferred_element_type=jnp.float32)
        + bc[...])
    hn = jnp.tanh(
        jnp.dot(nf[...], wn_[0:128], preferred_element_type=jnp.float32)
        + jnp.dot(nmean, wn_[128:256], preferred_element_type=jnp.float32)
        + jnp.dot(nmx, wn_[256:384], preferred_element_type=jnp.float32)
        + bn[...])
    u_out[...] = jnp.dot(hn, wu[...], preferred_element_type=jnp.float32)
    tnet_out[...] = (jnp.dot(hn, wt[...], preferred_element_type=jnp.float32)
                     + bt_bias[...])
    s_out[...] = (jnp.dot(hc, ws[...], preferred_element_type=jnp.float32)
                  + bs_bias[...])


def _d1(cf, csum, cmax, ccnt, nf, nsum, nmax, ncnt,
        wc, wn, wu, ws, wt, bc, bn, bs_bias, bt_bias):
    row = pl.BlockSpec((DB, 128), lambda i: (i, 0))
    row1 = pl.BlockSpec((DB, 1), lambda i: (i, 0))
    row16 = pl.BlockSpec((DB, 16), lambda i: (i, 0))
    full = lambda shape: pl.BlockSpec(shape, lambda i: tuple(0 for _ in shape))
    return pl.pallas_call(
        _d1_body,
        grid=(DGRID,),
        in_specs=[row, row, row, row1, row, row, row, row1,
                  full((384, 128)), full((384, 128)), full((128, 16)),
                  full((128, 16)), full((128, 16)), full((1, 128)),
                  full((1, 128)), full((1, 16)), full((1, 16))],
        out_specs=[row16, row16, row16],
        out_shape=[_f32((NPAD, 16)), _f32((NPAD, 16)), _f32((NPAD, 16))],
    )(cf, csum, cmax, ccnt, nf, nsum, nmax, ncnt,
      wc, wn, wu, ws, wt, bc, bn, bs_bias, bt_bias)


PB = 20480
PGRID = EPIN // PB


def _p1_body(pf, wp, bp, we, be, ew_out):
    hp = jnp.tanh(jnp.dot(pf[...], wp[...],
                          preferred_element_type=jnp.float32) + bp[...])
    ew_out[...] = jnp.tanh(jnp.dot(hp, we[...],
                                   preferred_element_type=jnp.float32)
                           + be[...])


def _p1(pf, wp, bp, we, be):
    full = lambda shape: pl.BlockSpec(shape, lambda i: tuple(0 for _ in shape))
    return pl.pallas_call(
        _p1_body,
        grid=(PGRID,),
        in_specs=[pl.BlockSpec((PB, 16), lambda i: (i, 0)),
                  full((16, 16)), full((1, 16)), full((16, 1)), full((1, 1))],
        out_specs=pl.BlockSpec((PB, 1), lambda i: (i, 0)),
        out_shape=_f32((EPIN, 1)),
    )(pf, wp, bp, we, be)


def _d2_body(s_in, w0, w1, cnt, size, tcell_out):
    t = s_in[...] + (w0[...] + w1[...]) / jnp.maximum(cnt[...], 1.0)
    tcell_out[...] = jnp.concatenate(
        [t[:, 0:5], size[...], jnp.zeros((DB, 9), jnp.float32)], axis=1)


def _d2(s_in, w0, w1, cnt, size):
    row16 = pl.BlockSpec((DB, 16), lambda i: (i, 0))
    return pl.pallas_call(
        _d2_body,
        grid=(DGRID,),
        in_specs=[row16, row16, row16, pl.BlockSpec((DB, 1), lambda i: (i, 0)),
                  pl.BlockSpec((DB, 2), lambda i: (i, 0))],
        out_specs=row16,
        out_shape=_f32((NPAD, 16)),
    )(s_in, w0, w1, cnt, size)


def kernel(cell_feat, net_feat, pin_feat, cell_size, pin_cell, pin_net,
           fathers, sons, grandfathers, fs_nets, gf_nets,
           W_cell, b_cell, W_net, b_net, W_pin, b_pin, W_ew, b_ew,
           W_self, W_neigh, b_sage, W_dis, b_dis, W_def, b_def):
    f32 = jnp.float32
    i32 = jnp.int32

    padn = NPAD - NCELL
    cf_p = jnp.concatenate([cell_feat, jnp.zeros((padn, 128), f32)])
    nf_p = jnp.concatenate([net_feat, jnp.zeros((padn, 128), f32)])
    size_p = jnp.concatenate([cell_size, jnp.zeros((padn, 2), f32)])

    padp = EPIN - NPIN
    ar = jnp.arange(padp, dtype=i32)
    pad_dst = NCELL + (ar % padn)
    pc_p = jnp.concatenate([pin_cell.astype(i32), pad_dst])
    pn_p = jnp.concatenate([pin_net.astype(i32), pad_dst])
    pf_p = jnp.concatenate([pin_feat, jnp.zeros((padp, 16), f32)])

    pn_u = pn_p.astype(jnp.uint32)
    pc_u = pc_p.astype(jnp.uint32)
    dsrc_net = lax.bitcast_convert_type(
        (pn_u << 16) | pc_u, jnp.int32).reshape(NCHUNK, CH)
    dsrc_cell = lax.bitcast_convert_type(
        (pc_u << 16) | pn_u, jnp.int32).reshape(NCHUNK, CH)

    padq = PPAIR - NPAIR
    arq = jnp.arange(padq, dtype=i32)
    padq_idx = arq % NCELL
    fa2 = jnp.concatenate([fathers.astype(i32), padq_idx]).reshape(-1, 128)
    so2 = jnp.concatenate([sons.astype(i32), padq_idx]).reshape(-1, 128)
    gf2 = jnp.concatenate([grandfathers.astype(i32), padq_idx]).reshape(-1, 128)
    fn2 = jnp.concatenate([fs_nets.astype(i32), padq_idx]).reshape(-1, 128)
    gn2 = jnp.concatenate([gf_nets.astype(i32), padq_idx]).reshape(-1, 128)

    Wd_f, Wd_s, Wd_n = W_dis[0:128], W_dis[128:256], W_dis[256:384]
    We_g, We_f, We_s = W_def[0:128], W_def[128:256], W_def[256:384]
    We_gn, We_fn = W_def[384:512], W_def[512:640]
    cols = [Wd_f, Wd_s, We_g, We_f, We_s]
    WU = jnp.concatenate([W_neigh @ w for w in cols], axis=1)
    WS = jnp.concatenate([W_self @ w for w in cols], axis=1)
    kb = jnp.concatenate([b_sage @ w for w in cols])
    z11 = jnp.zeros((128, 11), f32)
    WU16 = jnp.concatenate([WU, z11], axis=1)
    WS16 = jnp.concatenate([WS, z11], axis=1)
    WT16 = jnp.concatenate([Wd_n, We_gn, We_fn, jnp.zeros((128, 13), f32)],
                           axis=1)
    bs_bias = jnp.concatenate([kb, jnp.zeros((11,), f32)]).reshape(1, 16)
    bt_bias = jnp.concatenate([b_dis, b_def, jnp.zeros((14,), f32)]
                              ).reshape(1, 16)

    nsum, nmax, ncnt = _seg_call(dsrc_net, cf_p)
    csum, cmax, ccnt = _seg_call(dsrc_cell, nf_p)

    U, TNET, S = _d1(cf_p, csum, cmax, ccnt.reshape(NPAD, 1),
                     nf_p, nsum, nmax, ncnt.reshape(NPAD, 1),
                     W_cell, W_net, WU16, WS16, WT16,
                     b_cell.reshape(1, 128), b_net.reshape(1, 128),
                     bs_bias, bt_bias)
    ew = _p1(pf_p, W_pin, b_pin.reshape(1, 16), W_ew, b_ew.reshape(1, 1))
    ew_masked = jnp.where(jnp.arange(EPIN, dtype=i32)[:, None] < NPIN,
                          ew, 0.0)

    wacc = _wseg_call(pn_p.reshape(-1, 128), pc_p.reshape(-1, 128),
                      ew_masked.reshape(-1, 128), U)[0]

    TCELL = _d2(S, wacc[0], wacc[1], ccnt.reshape(NPAD, 1), size_p)
    d_arr = TCELL[:, 2]
    g_arr = TNET[:, 1]

    o1, o2 = _pair_call(fa2, so2, gf2, fn2, gn2, TCELL, TNET, d_arr, g_arr)
    edge_dis = o1.reshape(-1)[:NPAIR]
    edge_deflect = o2.reshape(-1)[:NPAIR]
    return (edge_dis, edge_deflect)
